# baseline (speedup 1.0000x reference)
import functools

import jax
import jax.numpy as jnp
from jax import lax
from jax.experimental import pallas as pl
from jax.experimental.pallas import tpu as pltpu
from jax.experimental.pallas import tpu_sc as plsc

NC = 2
NS = 16
NW = NC * NS


def _sc_mesh():
    return plsc.VectorSubcoreMesh(
        core_axis_name="c", subcore_axis_name="s",
        num_cores=NC, num_subcores=NS)


def _make_hist(E, MDEG, K):
    EPW = E // NW
    nch = EPW // K
    RPT = MDEG // NS

    @functools.partial(
        pl.kernel,
        out_type=jax.ShapeDtypeStruct((NC * MDEG,), jnp.float32),
        mesh=_sc_mesh(),
        scratch_types=[
            pltpu.VMEM_SHARED((MDEG,), jnp.float32),
            pltpu.VMEM((nch, K), jnp.int32),
            pltpu.VMEM((K,), jnp.float32),
            pltpu.VMEM((RPT,), jnp.float32),
        ],
    )
    def hist(dst_hbm, out_hbm, deg_sh, dste, ones_v, zbuf):
        c = lax.axis_index("c")
        s = lax.axis_index("s")
        w = c * NS + s
        for i in range(RPT // 16):
            zbuf[pl.ds(i * 16, 16)] = jnp.zeros((16,), jnp.float32)
        for i in range(K // 16):
            ones_v[pl.ds(i * 16, 16)] = jnp.ones((16,), jnp.float32)
        pltpu.sync_copy(zbuf, deg_sh.at[pl.ds(s * RPT, RPT)])
        pltpu.sync_copy(dst_hbm.at[w], dste)
        plsc.subcore_barrier()

        def body(j, carry):
            pltpu.sync_copy(ones_v, deg_sh.at[dste.at[j]], add=True)
            return carry

        lax.fori_loop(0, nch, body, 0)
        plsc.subcore_barrier()
        pltpu.sync_copy(deg_sh.at[pl.ds(s * RPT, RPT)],
                        out_hbm.at[pl.ds(c * MDEG + s * RPT, RPT)])

    return hist


def _make_scatter(NPAD, D, E, K, GB):
    EPW = E // NW
    nch = EPW // K
    nblk = nch // GB
    RPT = NPAD // NS
    assert nch % GB == 0 and nblk % 2 == 0 and GB % 2 == 0

    @functools.partial(
        pl.kernel,
        out_type=jax.ShapeDtypeStruct((NC * NPAD, D), jnp.float32),
        mesh=_sc_mesh(),
        scratch_types=[
            pltpu.VMEM_SHARED((NPAD, D), jnp.float32),
            pltpu.VMEM((GB, 2, K), jnp.int32),
            pltpu.VMEM((GB, 2, K), jnp.int32),
            pltpu.VMEM((K, D), jnp.float32),
            pltpu.VMEM((K, D), jnp.float32),
            pltpu.SemaphoreType.DMA,
            pltpu.SemaphoreType.DMA,
            pltpu.SemaphoreType.DMA,
            pltpu.SemaphoreType.DMA,
        ],
    )
    def scat(g_hbm, edg_hbm, zeros_hbm, out_hbm,
             acc_sh, ixa, ixb, rows_a, rows_b, sem_a, sem_b, sem_ia, sem_ib):
        c = lax.axis_index("c")
        s = lax.axis_index("s")
        w = c * NS + s
        ZR = zeros_hbm.shape[0]
        for k in range(RPT // ZR):
            pltpu.async_copy(zeros_hbm,
                             acc_sh.at[pl.ds(s * RPT + k * ZR, ZR)], sem_a)
        for k in range(RPT // ZR):
            pltpu.make_async_copy(
                zeros_hbm, acc_sh.at[pl.ds(s * RPT + k * ZR, ZR)],
                sem_a).wait()
        plsc.subcore_barrier()

        def stage(b, buf, sem):
            return pltpu.async_copy(
                edg_hbm.at[w, pl.ds(b * GB, GB)], buf, sem)

        def wait_stage(buf, sem):
            pltpu.make_async_copy(edg_hbm.at[w, pl.ds(0, GB)], buf, sem).wait()

        def gather(ix, t, buf, sem):
            pltpu.async_copy(g_hbm.at[ix.at[t, 0]], buf, sem)

        def wait_rows(buf, sem):
            pltpu.make_async_copy(g_hbm.at[ixa.at[0, 0]], buf, sem).wait()

        def scatter(ix, t, buf):
            pltpu.sync_copy(buf, acc_sh.at[ix.at[t, 1]], add=True)

        def block(ix, nxt_ix, nxt_sem, has_next):
            def pair(ti, carry):
                t = 2 * ti
                wait_rows(rows_a, sem_a)
                gather(ix, t + 1, rows_b, sem_b)
                scatter(ix, t, rows_a)
                wait_rows(rows_b, sem_b)
                gather(ix, t + 2, rows_a, sem_a)
                scatter(ix, t + 1, rows_b)
                return carry

            lax.fori_loop(0, GB // 2 - 1, pair, 0)
            t = GB - 2
            wait_rows(rows_a, sem_a)
            gather(ix, t + 1, rows_b, sem_b)
            scatter(ix, t, rows_a)
            wait_rows(rows_b, sem_b)

            @pl.when(has_next)
            def _():
                wait_stage(nxt_ix, nxt_sem)
                gather(nxt_ix, 0, rows_a, sem_a)

            scatter(ix, t + 1, rows_b)

        stage(0, ixa, sem_ia).wait()
        stage(1, ixb, sem_ib)
        gather(ixa, 0, rows_a, sem_a)

        def outer2(bi, carry):
            b0 = 2 * bi
            block(ixa, ixb, sem_ib, b0 + 1 < nblk)

            @pl.when(b0 + 2 < nblk)
            def _():
                stage(b0 + 2, ixa, sem_ia)

            block(ixb, ixa, sem_ia, b0 + 2 < nblk)

            @pl.when(b0 + 3 < nblk)
            def _():
                stage(b0 + 3, ixb, sem_ib)

            return carry

        lax.fori_loop(0, nblk // 2, outer2, 0)
        plsc.subcore_barrier()
        pltpu.sync_copy(acc_sh.at[pl.ds(s * RPT, RPT)],
                        out_hbm.at[pl.ds(c * NPAD + s * RPT, RPT)])

    return scat


def _matmul(x, W):
    N, Din = x.shape
    Dout = W.shape[1]
    BN = 1000

    def body(x_ref, w_ref, h_ref):
        h_ref[...] = jnp.dot(x_ref[...], w_ref[...],
                             preferred_element_type=jnp.float32)

    return pl.pallas_call(
        body,
        grid=(N // BN,),
        in_specs=[
            pl.BlockSpec((BN, Din), lambda i: (i, 0)),
            pl.BlockSpec((Din, Dout), lambda i: (0, 0)),
        ],
        out_specs=pl.BlockSpec((BN, Dout), lambda i: (i, 0)),
        out_shape=jax.ShapeDtypeStruct((N, Dout), jnp.float32),
    )(x, W)


def _scale(h, d0, d1):
    N, Dout = h.shape
    BN = 1000

    def body(h_ref, d0_ref, d1_ref, g_ref, dinv_ref):
        dinv = lax.rsqrt(d0_ref[...] + d1_ref[...] + 1.0)
        g_ref[...] = h_ref[...] * dinv
        dinv_ref[...] = dinv

    return pl.pallas_call(
        body,
        grid=(N // BN,),
        in_specs=[
            pl.BlockSpec((BN, Dout), lambda i: (i, 0)),
            pl.BlockSpec((BN, 1), lambda i: (i, 0)),
            pl.BlockSpec((BN, 1), lambda i: (i, 0)),
        ],
        out_specs=[
            pl.BlockSpec((BN, Dout), lambda i: (i, 0)),
            pl.BlockSpec((BN, 1), lambda i: (i, 0)),
        ],
        out_shape=[
            jax.ShapeDtypeStruct((N, Dout), jnp.float32),
            jax.ShapeDtypeStruct((N, 1), jnp.float32),
        ],
    )(h, d0, d1)


def _final(acc, g, dinv, b2d):
    N = g.shape[0]
    D = g.shape[1]
    BN = 1000

    def body(a_ref, g_ref, dinv_ref, b_ref, o_ref):
        o_ref[...] = (dinv_ref[...] * (a_ref[0] + a_ref[1] + g_ref[...])
                      + b_ref[...])

    return pl.pallas_call(
        body,
        grid=(N // BN,),
        in_specs=[
            pl.BlockSpec((2, BN, D), lambda i: (0, i, 0)),
            pl.BlockSpec((BN, D), lambda i: (i, 0)),
            pl.BlockSpec((BN, 1), lambda i: (i, 0)),
            pl.BlockSpec((1, D), lambda i: (0, 0)),
        ],
        out_specs=pl.BlockSpec((BN, D), lambda i: (i, 0)),
        out_shape=jax.ShapeDtypeStruct((N, D), jnp.float32),
    )(acc, g, dinv, b2d)


def kernel(x, edge_index, t_embed, W, b):
    N, Din = x.shape
    Dout = W.shape[1]
    E = edge_index.shape[1]
    src = edge_index[0]
    dst = edge_index[1]

    NPAD = 10240
    KH = 80
    K = 125
    GB = 8

    nch = E // NW // K
    edg = jnp.stack(
        [src.reshape(NW, nch, K), dst.reshape(NW, nch, K)], axis=2)
    dst3dh = dst.reshape(NW, E // NW // KH, KH)

    h = _matmul(x, W)
    degp = _make_hist(E, NPAD, KH)(dst3dh)
    d0 = degp[:N].reshape(N, 1)
    d1 = degp[NPAD:NPAD + N].reshape(N, 1)

    g, dinv = _scale(h, d0, d1)

    zeros2d = jnp.zeros((128, Dout), jnp.float32)
    acc = _make_scatter(NPAD, Dout, E, K, GB)(g, edg, zeros2d)
    acc = acc.reshape(NC, NPAD, Dout)

    out = _final(acc, g, dinv, b.reshape(1, Dout))
    return (out, edge_index, t_embed)


# speedup vs baseline: 1.0275x; 1.0275x over previous
import functools

import jax
import jax.numpy as jnp
from jax import lax
from jax.experimental import pallas as pl
from jax.experimental.pallas import tpu as pltpu
from jax.experimental.pallas import tpu_sc as plsc

NC = 2
NS = 16
NW = NC * NS


def _sc_mesh():
    return plsc.VectorSubcoreMesh(
        core_axis_name="c", subcore_axis_name="s",
        num_cores=NC, num_subcores=NS)


def _make_hist(E, MDEG, K):
    EPW = E // NW
    nch = EPW // K
    RPT = MDEG // NS

    @functools.partial(
        pl.kernel,
        out_type=jax.ShapeDtypeStruct((NC * MDEG,), jnp.float32),
        mesh=_sc_mesh(),
        scratch_types=[
            pltpu.VMEM_SHARED((MDEG,), jnp.float32),
            pltpu.VMEM((nch, K), jnp.int32),
            pltpu.VMEM((K,), jnp.float32),
            pltpu.VMEM((RPT,), jnp.float32),
        ],
    )
    def hist(dst_hbm, out_hbm, deg_sh, dste, ones_v, zbuf):
        c = lax.axis_index("c")
        s = lax.axis_index("s")
        w = c * NS + s
        for i in range(RPT // 16):
            zbuf[pl.ds(i * 16, 16)] = jnp.zeros((16,), jnp.float32)
        for i in range(K // 16):
            ones_v[pl.ds(i * 16, 16)] = jnp.ones((16,), jnp.float32)
        pltpu.sync_copy(zbuf, deg_sh.at[pl.ds(s * RPT, RPT)])
        pltpu.sync_copy(dst_hbm.at[w], dste)
        plsc.subcore_barrier()

        def body(j, carry):
            pltpu.sync_copy(ones_v, deg_sh.at[dste.at[j]], add=True)
            return carry

        lax.fori_loop(0, nch, body, 0)
        plsc.subcore_barrier()
        pltpu.sync_copy(deg_sh.at[pl.ds(s * RPT, RPT)],
                        out_hbm.at[pl.ds(c * MDEG + s * RPT, RPT)])

    return hist


def _make_scatter(NPAD, D, E, K, GB):
    EPW = E // NW
    nch = EPW // K
    nblk = nch // GB
    RPT = NPAD // NS
    assert nch % GB == 0 and nblk % 2 == 0 and GB % 2 == 0

    @functools.partial(
        pl.kernel,
        out_type=jax.ShapeDtypeStruct((NC * NPAD, D), jnp.float32),
        mesh=_sc_mesh(),
        scratch_types=[
            pltpu.VMEM_SHARED((NPAD, D), jnp.float32),
            pltpu.VMEM((GB, 2, K), jnp.int32),
            pltpu.VMEM((GB, 2, K), jnp.int32),
            pltpu.VMEM((K, D), jnp.float32),
            pltpu.VMEM((K, D), jnp.float32),
            pltpu.SemaphoreType.DMA,
            pltpu.SemaphoreType.DMA,
            pltpu.SemaphoreType.DMA,
            pltpu.SemaphoreType.DMA,
        ],
    )
    def scat(g_hbm, edg_hbm, zeros_hbm, out_hbm,
             acc_sh, ixa, ixb, rows_a, rows_b, sem_a, sem_b, sem_ia, sem_ib):
        c = lax.axis_index("c")
        s = lax.axis_index("s")
        w = c * NS + s
        ZR = zeros_hbm.shape[0]
        for k in range(RPT // ZR):
            pltpu.async_copy(zeros_hbm,
                             acc_sh.at[pl.ds(s * RPT + k * ZR, ZR)], sem_a)
        for k in range(RPT // ZR):
            pltpu.make_async_copy(
                zeros_hbm, acc_sh.at[pl.ds(s * RPT + k * ZR, ZR)],
                sem_a).wait()
        plsc.subcore_barrier()

        def stage(b, buf, sem):
            return pltpu.async_copy(
                edg_hbm.at[w, pl.ds(b * GB, GB)], buf, sem)

        def wait_stage(buf, sem):
            pltpu.make_async_copy(edg_hbm.at[w, pl.ds(0, GB)], buf, sem).wait()

        def gather(ix, t, buf, sem):
            pltpu.async_copy(g_hbm.at[ix.at[t, 0]], buf, sem)

        def wait_rows(buf, sem):
            pltpu.make_async_copy(g_hbm.at[ixa.at[0, 0]], buf, sem).wait()

        def scatter(ix, t, buf):
            pltpu.sync_copy(buf, acc_sh.at[ix.at[t, 1]], add=True)

        def block(ix, nxt_ix, nxt_sem, has_next):
            def pair(ti, carry):
                t = 2 * ti
                wait_rows(rows_a, sem_a)
                gather(ix, t + 1, rows_b, sem_b)
                scatter(ix, t, rows_a)
                wait_rows(rows_b, sem_b)
                gather(ix, t + 2, rows_a, sem_a)
                scatter(ix, t + 1, rows_b)
                return carry

            lax.fori_loop(0, GB // 2 - 1, pair, 0)
            t = GB - 2
            wait_rows(rows_a, sem_a)
            gather(ix, t + 1, rows_b, sem_b)
            scatter(ix, t, rows_a)
            wait_rows(rows_b, sem_b)

            @pl.when(has_next)
            def _():
                wait_stage(nxt_ix, nxt_sem)
                gather(nxt_ix, 0, rows_a, sem_a)

            scatter(ix, t + 1, rows_b)

        stage(0, ixa, sem_ia).wait()
        stage(1, ixb, sem_ib)
        gather(ixa, 0, rows_a, sem_a)

        def outer2(bi, carry):
            b0 = 2 * bi
            block(ixa, ixb, sem_ib, b0 + 1 < nblk)

            @pl.when(b0 + 2 < nblk)
            def _():
                stage(b0 + 2, ixa, sem_ia)

            block(ixb, ixa, sem_ia, b0 + 2 < nblk)

            @pl.when(b0 + 3 < nblk)
            def _():
                stage(b0 + 3, ixb, sem_ib)

            return carry

        lax.fori_loop(0, nblk // 2, outer2, 0)
        plsc.subcore_barrier()
        pltpu.sync_copy(acc_sh.at[pl.ds(s * RPT, RPT)],
                        out_hbm.at[pl.ds(c * NPAD + s * RPT, RPT)])

    return scat


def _matmul_scale(x, W, d0, d1):
    N, Din = x.shape
    Dout = W.shape[1]
    BN = 1000

    def body(x_ref, w_ref, d0_ref, d1_ref, g_ref, dinv_ref):
        dinv = lax.rsqrt(d0_ref[...] + d1_ref[...] + 1.0)
        h = jnp.dot(x_ref[...], w_ref[...],
                    preferred_element_type=jnp.float32)
        g_ref[...] = h * dinv
        dinv_ref[...] = dinv

    return pl.pallas_call(
        body,
        grid=(N // BN,),
        in_specs=[
            pl.BlockSpec((BN, Din), lambda i: (i, 0)),
            pl.BlockSpec((Din, Dout), lambda i: (0, 0)),
            pl.BlockSpec((BN, 1), lambda i: (i, 0)),
            pl.BlockSpec((BN, 1), lambda i: (i, 0)),
        ],
        out_specs=[
            pl.BlockSpec((BN, Dout), lambda i: (i, 0)),
            pl.BlockSpec((BN, 1), lambda i: (i, 0)),
        ],
        out_shape=[
            jax.ShapeDtypeStruct((N, Dout), jnp.float32),
            jax.ShapeDtypeStruct((N, 1), jnp.float32),
        ],
    )(x, W, d0, d1)


def _final(acc, g, dinv, b2d):
    N = g.shape[0]
    D = g.shape[1]
    BN = 1000

    def body(a_ref, g_ref, dinv_ref, b_ref, o_ref):
        o_ref[...] = (dinv_ref[...] * (a_ref[0] + a_ref[1] + g_ref[...])
                      + b_ref[...])

    return pl.pallas_call(
        body,
        grid=(N // BN,),
        in_specs=[
            pl.BlockSpec((2, BN, D), lambda i: (0, i, 0)),
            pl.BlockSpec((BN, D), lambda i: (i, 0)),
            pl.BlockSpec((BN, 1), lambda i: (i, 0)),
            pl.BlockSpec((1, D), lambda i: (0, 0)),
        ],
        out_specs=pl.BlockSpec((BN, D), lambda i: (i, 0)),
        out_shape=jax.ShapeDtypeStruct((N, D), jnp.float32),
    )(acc, g, dinv, b2d)


def kernel(x, edge_index, t_embed, W, b):
    N, Din = x.shape
    Dout = W.shape[1]
    E = edge_index.shape[1]
    src = edge_index[0]
    dst = edge_index[1]

    NPAD = 10240
    KH = 80
    K = 125
    GB = 8

    nch = E // NW // K
    edg = jnp.stack(
        [src.reshape(NW, nch, K), dst.reshape(NW, nch, K)], axis=2)
    dst3dh = dst.reshape(NW, E // NW // KH, KH)

    degp = _make_hist(E, NPAD, KH)(dst3dh)
    d0 = degp[:N].reshape(N, 1)
    d1 = degp[NPAD:NPAD + N].reshape(N, 1)

    g, dinv = _matmul_scale(x, W, d0, d1)

    zeros2d = jnp.zeros((128, Dout), jnp.float32)
    acc = _make_scatter(NPAD, Dout, E, K, GB)(g, edg, zeros2d)
    acc = acc.reshape(NC, NPAD, Dout)

    out = _final(acc, g, dinv, b.reshape(1, Dout))
    return (out, edge_index, t_embed)


# speedup vs baseline: 1.0514x; 1.0233x over previous
import functools

import jax
import jax.numpy as jnp
from jax import lax
from jax.experimental import pallas as pl
from jax.experimental.pallas import tpu as pltpu
from jax.experimental.pallas import tpu_sc as plsc

NC = 2
NS = 16
NW = NC * NS


def _sc_mesh():
    return plsc.VectorSubcoreMesh(
        core_axis_name="c", subcore_axis_name="s",
        num_cores=NC, num_subcores=NS)


def _make_hist(E, MDEG, K):
    EPW = E // NW
    nch = EPW // K
    RPT = MDEG // NS

    @functools.partial(
        pl.kernel,
        out_type=jax.ShapeDtypeStruct((NC * MDEG,), jnp.float32),
        mesh=_sc_mesh(),
        scratch_types=[
            pltpu.VMEM_SHARED((MDEG,), jnp.float32),
            pltpu.VMEM((nch, K), jnp.int32),
            pltpu.VMEM((K,), jnp.float32),
            pltpu.VMEM((RPT,), jnp.float32),
        ],
    )
    def hist(dst_hbm, out_hbm, deg_sh, dste, ones_v, zbuf):
        c = lax.axis_index("c")
        s = lax.axis_index("s")
        w = c * NS + s
        for i in range(RPT // 16):
            zbuf[pl.ds(i * 16, 16)] = jnp.zeros((16,), jnp.float32)
        for i in range(K // 16):
            ones_v[pl.ds(i * 16, 16)] = jnp.ones((16,), jnp.float32)
        pltpu.sync_copy(zbuf, deg_sh.at[pl.ds(s * RPT, RPT)])
        pltpu.sync_copy(dst_hbm.at[w], dste)
        plsc.subcore_barrier()

        def body(j, carry):
            pltpu.sync_copy(ones_v, deg_sh.at[dste.at[j]], add=True)
            return carry

        lax.fori_loop(0, nch, body, 0)
        plsc.subcore_barrier()
        pltpu.sync_copy(deg_sh.at[pl.ds(s * RPT, RPT)],
                        out_hbm.at[pl.ds(c * MDEG + s * RPT, RPT)])

    return hist


def _make_scatter(NPAD, D, E, K, GB):
    EPW = E // NW
    nch = EPW // K
    nblk = nch // GB
    RPT = NPAD // NS
    assert nch % GB == 0 and nblk % 2 == 0 and GB % 2 == 0

    @functools.partial(
        pl.kernel,
        out_type=jax.ShapeDtypeStruct((NC * NPAD, D), jnp.float32),
        mesh=_sc_mesh(),
        scratch_types=[
            pltpu.VMEM_SHARED((NPAD, D), jnp.float32),
            pltpu.VMEM((GB, 2, K), jnp.int32),
            pltpu.VMEM((GB, 2, K), jnp.int32),
            pltpu.VMEM((K, D), jnp.float32),
            pltpu.VMEM((K, D), jnp.float32),
            pltpu.SemaphoreType.DMA,
            pltpu.SemaphoreType.DMA,
            pltpu.SemaphoreType.DMA,
            pltpu.SemaphoreType.DMA,
        ],
    )
    def scat(g_hbm, edg_hbm, zeros_hbm, out_hbm,
             acc_sh, ixa, ixb, rows_a, rows_b, sem_a, sem_b, sem_ia, sem_ib):
        c = lax.axis_index("c")
        s = lax.axis_index("s")
        w = c * NS + s
        pltpu.sync_copy(zeros_hbm.at[pl.ds(s * RPT, RPT)],
                        acc_sh.at[pl.ds(s * RPT, RPT)])
        plsc.subcore_barrier()

        def stage(b, buf, sem):
            return pltpu.async_copy(
                edg_hbm.at[w, pl.ds(b * GB, GB)], buf, sem)

        def wait_stage(buf, sem):
            pltpu.make_async_copy(edg_hbm.at[w, pl.ds(0, GB)], buf, sem).wait()

        def gather(ix, t, buf, sem):
            pltpu.async_copy(g_hbm.at[ix.at[t, 0]], buf, sem)

        def wait_rows(buf, sem):
            pltpu.make_async_copy(g_hbm.at[ixa.at[0, 0]], buf, sem).wait()

        def scatter(ix, t, buf):
            pltpu.sync_copy(buf, acc_sh.at[ix.at[t, 1]], add=True)

        def block(ix, nxt_ix, nxt_sem, has_next):
            def pair(ti, carry):
                t = 2 * ti
                wait_rows(rows_a, sem_a)
                gather(ix, t + 1, rows_b, sem_b)
                scatter(ix, t, rows_a)
                wait_rows(rows_b, sem_b)
                gather(ix, t + 2, rows_a, sem_a)
                scatter(ix, t + 1, rows_b)
                return carry

            lax.fori_loop(0, GB // 2 - 1, pair, 0)
            t = GB - 2
            wait_rows(rows_a, sem_a)
            gather(ix, t + 1, rows_b, sem_b)
            scatter(ix, t, rows_a)
            wait_rows(rows_b, sem_b)

            @pl.when(has_next)
            def _():
                wait_stage(nxt_ix, nxt_sem)
                gather(nxt_ix, 0, rows_a, sem_a)

            scatter(ix, t + 1, rows_b)

        stage(0, ixa, sem_ia).wait()
        stage(1, ixb, sem_ib)
        gather(ixa, 0, rows_a, sem_a)

        def outer2(bi, carry):
            b0 = 2 * bi
            block(ixa, ixb, sem_ib, b0 + 1 < nblk)

            @pl.when(b0 + 2 < nblk)
            def _():
                stage(b0 + 2, ixa, sem_ia)

            block(ixb, ixa, sem_ia, b0 + 2 < nblk)

            @pl.when(b0 + 3 < nblk)
            def _():
                stage(b0 + 3, ixb, sem_ib)

            return carry

        lax.fori_loop(0, nblk // 2, outer2, 0)
        plsc.subcore_barrier()
        pltpu.sync_copy(acc_sh.at[pl.ds(s * RPT, RPT)],
                        out_hbm.at[pl.ds(c * NPAD + s * RPT, RPT)])

    return scat


def _matmul_scale(x, W, d0, d1):
    N, Din = x.shape
    Dout = W.shape[1]
    BN = 1000

    def body(x_ref, w_ref, d0_ref, d1_ref, g_ref, dinv_ref):
        dinv = lax.rsqrt(d0_ref[...] + d1_ref[...] + 1.0)
        h = jnp.dot(x_ref[...], w_ref[...],
                    preferred_element_type=jnp.float32)
        g_ref[...] = h * dinv
        dinv_ref[...] = dinv

    return pl.pallas_call(
        body,
        grid=(N // BN,),
        in_specs=[
            pl.BlockSpec((BN, Din), lambda i: (i, 0)),
            pl.BlockSpec((Din, Dout), lambda i: (0, 0)),
            pl.BlockSpec((BN, 1), lambda i: (i, 0)),
            pl.BlockSpec((BN, 1), lambda i: (i, 0)),
        ],
        out_specs=[
            pl.BlockSpec((BN, Dout), lambda i: (i, 0)),
            pl.BlockSpec((BN, 1), lambda i: (i, 0)),
        ],
        out_shape=[
            jax.ShapeDtypeStruct((N, Dout), jnp.float32),
            jax.ShapeDtypeStruct((N, 1), jnp.float32),
        ],
    )(x, W, d0, d1)


def _final(acc, g, dinv, b2d):
    N = g.shape[0]
    D = g.shape[1]
    BN = 1000

    def body(a_ref, g_ref, dinv_ref, b_ref, o_ref):
        o_ref[...] = (dinv_ref[...] * (a_ref[0] + a_ref[1] + g_ref[...])
                      + b_ref[...])

    return pl.pallas_call(
        body,
        grid=(N // BN,),
        in_specs=[
            pl.BlockSpec((2, BN, D), lambda i: (0, i, 0)),
            pl.BlockSpec((BN, D), lambda i: (i, 0)),
            pl.BlockSpec((BN, 1), lambda i: (i, 0)),
            pl.BlockSpec((1, D), lambda i: (0, 0)),
        ],
        out_specs=pl.BlockSpec((BN, D), lambda i: (i, 0)),
        out_shape=jax.ShapeDtypeStruct((N, D), jnp.float32),
    )(acc, g, dinv, b2d)


def kernel(x, edge_index, t_embed, W, b):
    N, Din = x.shape
    Dout = W.shape[1]
    E = edge_index.shape[1]
    src = edge_index[0]
    dst = edge_index[1]

    NPAD = 10240
    KH = 80
    K = 125
    GB = 8

    nch = E // NW // K
    edg = jnp.stack(
        [src.reshape(NW, nch, K), dst.reshape(NW, nch, K)], axis=2)
    dst3dh = dst.reshape(NW, E // NW // KH, KH)

    degp = _make_hist(E, NPAD, KH)(dst3dh)
    d0 = degp[:N].reshape(N, 1)
    d1 = degp[NPAD:NPAD + N].reshape(N, 1)

    g, dinv = _matmul_scale(x, W, d0, d1)

    zeros2d = jnp.zeros((NPAD, Dout), jnp.float32)
    acc = _make_scatter(NPAD, Dout, E, K, GB)(g, edg, zeros2d)
    acc = acc.reshape(NC, NPAD, Dout)

    out = _final(acc, g, dinv, b.reshape(1, Dout))
    return (out, edge_index, t_embed)


# speedup vs baseline: 1.0581x; 1.0063x over previous
import functools

import jax
import jax.numpy as jnp
from jax import lax
from jax.experimental import pallas as pl
from jax.experimental.pallas import tpu as pltpu
from jax.experimental.pallas import tpu_sc as plsc

NC = 2
NS = 16
NW = NC * NS


def _sc_mesh():
    return plsc.VectorSubcoreMesh(
        core_axis_name="c", subcore_axis_name="s",
        num_cores=NC, num_subcores=NS)


def _make_hist(E, MDEG, K):
    EPW = E // NW
    nch = EPW // K
    RPT = MDEG // NS

    @functools.partial(
        pl.kernel,
        out_type=jax.ShapeDtypeStruct((NC * MDEG,), jnp.float32),
        mesh=_sc_mesh(),
        scratch_types=[
            pltpu.VMEM_SHARED((MDEG,), jnp.float32),
            pltpu.VMEM((nch, K), jnp.int32),
            pltpu.VMEM((K,), jnp.float32),
            pltpu.VMEM((RPT,), jnp.float32),
        ],
    )
    def hist(dst_hbm, out_hbm, deg_sh, dste, ones_v, zbuf):
        c = lax.axis_index("c")
        s = lax.axis_index("s")
        w = c * NS + s
        for i in range(RPT // 16):
            zbuf[pl.ds(i * 16, 16)] = jnp.zeros((16,), jnp.float32)
        for i in range(K // 16):
            ones_v[pl.ds(i * 16, 16)] = jnp.ones((16,), jnp.float32)
        pltpu.sync_copy(zbuf, deg_sh.at[pl.ds(s * RPT, RPT)])
        pltpu.sync_copy(dst_hbm.at[w], dste)
        plsc.subcore_barrier()

        def body(j, carry):
            pltpu.sync_copy(ones_v, deg_sh.at[dste.at[j]], add=True)
            return carry

        lax.fori_loop(0, nch, body, 0)
        plsc.subcore_barrier()
        pltpu.sync_copy(deg_sh.at[pl.ds(s * RPT, RPT)],
                        out_hbm.at[pl.ds(c * MDEG + s * RPT, RPT)])

    return hist


def _make_scatter(NPAD, D, E, K, GB):
    EPW = E // NW
    nch = EPW // K
    nblk = nch // GB
    RPT = NPAD // NS
    assert nch % GB == 0 and nblk % 2 == 0 and GB % 2 == 0

    @functools.partial(
        pl.kernel,
        out_type=jax.ShapeDtypeStruct((NC * NPAD, D), jnp.float32),
        mesh=_sc_mesh(),
        scratch_types=[
            pltpu.VMEM_SHARED((NPAD, D), jnp.float32),
            pltpu.VMEM((GB, 2, K), jnp.int32),
            pltpu.VMEM((GB, 2, K), jnp.int32),
            pltpu.VMEM((K, D), jnp.float32),
            pltpu.VMEM((K, D), jnp.float32),
            pltpu.SemaphoreType.DMA,
            pltpu.SemaphoreType.DMA,
            pltpu.SemaphoreType.DMA,
            pltpu.SemaphoreType.DMA,
        ],
    )
    def scat(g_hbm, edg_hbm, out_hbm,
             acc_sh, ixa, ixb, rows_a, rows_b, sem_a, sem_b, sem_ia, sem_ib):
        c = lax.axis_index("c")
        s = lax.axis_index("s")
        w = c * NS + s
        pltpu.sync_copy(g_hbm.at[pl.ds(s * RPT, RPT)],
                        acc_sh.at[pl.ds(s * RPT, RPT)])
        plsc.subcore_barrier()

        def stage(b, buf, sem):
            return pltpu.async_copy(
                edg_hbm.at[w, pl.ds(b * GB, GB)], buf, sem)

        def wait_stage(buf, sem):
            pltpu.make_async_copy(edg_hbm.at[w, pl.ds(0, GB)], buf, sem).wait()

        def gather(ix, t, buf, sem):
            pltpu.async_copy(g_hbm.at[ix.at[t, 0]], buf, sem)

        def wait_rows(buf, sem):
            pltpu.make_async_copy(g_hbm.at[ixa.at[0, 0]], buf, sem).wait()

        def scatter(ix, t, buf):
            pltpu.sync_copy(buf, acc_sh.at[ix.at[t, 1]], add=True)

        def block(ix, nxt_ix, nxt_sem, has_next):
            def pair(ti, carry):
                t = 2 * ti
                wait_rows(rows_a, sem_a)
                gather(ix, t + 1, rows_b, sem_b)
                scatter(ix, t, rows_a)
                wait_rows(rows_b, sem_b)
                gather(ix, t + 2, rows_a, sem_a)
                scatter(ix, t + 1, rows_b)
                return carry

            lax.fori_loop(0, GB // 2 - 1, pair, 0)
            t = GB - 2
            wait_rows(rows_a, sem_a)
            gather(ix, t + 1, rows_b, sem_b)
            scatter(ix, t, rows_a)
            wait_rows(rows_b, sem_b)

            @pl.when(has_next)
            def _():
                wait_stage(nxt_ix, nxt_sem)
                gather(nxt_ix, 0, rows_a, sem_a)

            scatter(ix, t + 1, rows_b)

        stage(0, ixa, sem_ia).wait()
        stage(1, ixb, sem_ib)
        gather(ixa, 0, rows_a, sem_a)

        def outer2(bi, carry):
            b0 = 2 * bi
            block(ixa, ixb, sem_ib, b0 + 1 < nblk)

            @pl.when(b0 + 2 < nblk)
            def _():
                stage(b0 + 2, ixa, sem_ia)

            block(ixb, ixa, sem_ia, b0 + 2 < nblk)

            @pl.when(b0 + 3 < nblk)
            def _():
                stage(b0 + 3, ixb, sem_ib)

            return carry

        lax.fori_loop(0, nblk // 2, outer2, 0)
        plsc.subcore_barrier()
        pltpu.sync_copy(acc_sh.at[pl.ds(s * RPT, RPT)],
                        out_hbm.at[pl.ds(c * NPAD + s * RPT, RPT)])

    return scat


def _matmul_scale(x, W, d0, d1, NPAD):
    N, Din = x.shape
    Dout = W.shape[1]
    BN = 1000

    def body(x_ref, w_ref, d0_ref, d1_ref, g_ref, dinv_ref):
        dinv = lax.rsqrt(d0_ref[...] + d1_ref[...] + 1.0)
        h = jnp.dot(x_ref[...], w_ref[...],
                    preferred_element_type=jnp.float32)
        g_ref[...] = h * dinv
        dinv_ref[...] = dinv

    return pl.pallas_call(
        body,
        grid=(N // BN,),
        in_specs=[
            pl.BlockSpec((BN, Din), lambda i: (i, 0)),
            pl.BlockSpec((Din, Dout), lambda i: (0, 0)),
            pl.BlockSpec((BN, 1), lambda i: (i, 0)),
            pl.BlockSpec((BN, 1), lambda i: (i, 0)),
        ],
        out_specs=[
            pl.BlockSpec((BN, Dout), lambda i: (i, 0)),
            pl.BlockSpec((BN, 1), lambda i: (i, 0)),
        ],
        out_shape=[
            jax.ShapeDtypeStruct((NPAD, Dout), jnp.float32),
            jax.ShapeDtypeStruct((N, 1), jnp.float32),
        ],
    )(x, W, d0, d1)


def _final(acc, g, dinv, b2d, N):
    D = g.shape[1]
    BN = 1000

    def body(a_ref, g_ref, dinv_ref, b_ref, o_ref):
        o_ref[...] = (dinv_ref[...] * (a_ref[0] + a_ref[1] - g_ref[...])
                      + b_ref[...])

    return pl.pallas_call(
        body,
        grid=(N // BN,),
        in_specs=[
            pl.BlockSpec((2, BN, D), lambda i: (0, i, 0)),
            pl.BlockSpec((BN, D), lambda i: (i, 0)),
            pl.BlockSpec((BN, 1), lambda i: (i, 0)),
            pl.BlockSpec((1, D), lambda i: (0, 0)),
        ],
        out_specs=pl.BlockSpec((BN, D), lambda i: (i, 0)),
        out_shape=jax.ShapeDtypeStruct((N, D), jnp.float32),
    )(acc, g, dinv, b2d)


def kernel(x, edge_index, t_embed, W, b):
    N, Din = x.shape
    Dout = W.shape[1]
    E = edge_index.shape[1]
    src = edge_index[0]
    dst = edge_index[1]

    NPAD = 10240
    KH = 80
    K = 125
    GB = 8

    nch = E // NW // K
    edg = jnp.stack(
        [src.reshape(NW, nch, K), dst.reshape(NW, nch, K)], axis=2)
    dst3dh = dst.reshape(NW, E // NW // KH, KH)

    degp = _make_hist(E, NPAD, KH)(dst3dh)
    d0 = degp[:N].reshape(N, 1)
    d1 = degp[NPAD:NPAD + N].reshape(N, 1)

    g, dinv = _matmul_scale(x, W, d0, d1, NPAD)

    acc = _make_scatter(NPAD, Dout, E, K, GB)(g, edg)
    acc = acc.reshape(NC, NPAD, Dout)

    out = _final(acc, g, dinv, b.reshape(1, Dout), N)
    return (out, edge_index, t_embed)


# speedup vs baseline: 1.1082x; 1.0474x over previous
import functools

import jax
import jax.numpy as jnp
from jax import lax
from jax.experimental import pallas as pl
from jax.experimental.pallas import tpu as pltpu
from jax.experimental.pallas import tpu_sc as plsc

NC = 2
NS = 16
NW = NC * NS


def _sc_mesh():
    return plsc.VectorSubcoreMesh(
        core_axis_name="c", subcore_axis_name="s",
        num_cores=NC, num_subcores=NS)


def _make_hist(E, MDEG, K):
    EPW = E // NW
    nch = EPW // K
    RPT = MDEG // NS

    @functools.partial(
        pl.kernel,
        out_type=jax.ShapeDtypeStruct((NC * MDEG,), jnp.float32),
        mesh=_sc_mesh(),
        scratch_types=[
            pltpu.VMEM_SHARED((MDEG,), jnp.float32),
            pltpu.VMEM((nch, K), jnp.int32),
            pltpu.VMEM((K,), jnp.float32),
            pltpu.VMEM((RPT,), jnp.float32),
        ],
    )
    def hist(dst_hbm, out_hbm, deg_sh, dste, ones_v, zbuf):
        c = lax.axis_index("c")
        s = lax.axis_index("s")
        w = c * NS + s
        for i in range(RPT // 16):
            zbuf[pl.ds(i * 16, 16)] = jnp.zeros((16,), jnp.float32)
        for i in range(K // 16):
            ones_v[pl.ds(i * 16, 16)] = jnp.ones((16,), jnp.float32)
        pltpu.sync_copy(zbuf, deg_sh.at[pl.ds(s * RPT, RPT)])
        pltpu.sync_copy(dst_hbm.at[w], dste)
        plsc.subcore_barrier()

        def body(j, carry):
            pltpu.sync_copy(ones_v, deg_sh.at[dste.at[j]], add=True)
            return carry

        lax.fori_loop(0, nch, body, 0)
        plsc.subcore_barrier()
        pltpu.sync_copy(deg_sh.at[pl.ds(s * RPT, RPT)],
                        out_hbm.at[pl.ds(c * MDEG + s * RPT, RPT)])

    return hist


def _make_scatter(NPAD, D, E, K, GB):
    EPW = E // NW
    nch = EPW // K
    nblk = nch // GB
    RPT = NPAD // NS
    assert nch % GB == 0 and nblk % 2 == 0 and GB % 2 == 0

    @functools.partial(
        pl.kernel,
        out_type=jax.ShapeDtypeStruct((NC * NPAD, D), jnp.bfloat16),
        mesh=_sc_mesh(),
        compiler_params=pltpu.CompilerParams(use_tc_tiling_on_sc=False),
        scratch_types=[
            pltpu.VMEM_SHARED((NPAD, D), jnp.bfloat16),
            pltpu.VMEM((GB, 2, K), jnp.int32),
            pltpu.VMEM((GB, 2, K), jnp.int32),
            pltpu.VMEM((K, D), jnp.bfloat16),
            pltpu.VMEM((K, D), jnp.bfloat16),
            pltpu.SemaphoreType.DMA,
            pltpu.SemaphoreType.DMA,
            pltpu.SemaphoreType.DMA,
            pltpu.SemaphoreType.DMA,
        ],
    )
    def scat(gbf_hbm, edg_hbm, out_hbm,
             acc_sh, ixa, ixb, rows_a, rows_b, sem_a, sem_b, sem_ia, sem_ib):
        c = lax.axis_index("c")
        s = lax.axis_index("s")
        w = c * NS + s
        pltpu.sync_copy(gbf_hbm.at[pl.ds(s * RPT, RPT)],
                        acc_sh.at[pl.ds(s * RPT, RPT)])
        plsc.subcore_barrier()

        def stage(b, buf, sem):
            return pltpu.async_copy(
                edg_hbm.at[w, pl.ds(b * GB, GB)], buf, sem)

        def wait_stage(buf, sem):
            pltpu.make_async_copy(edg_hbm.at[w, pl.ds(0, GB)], buf, sem).wait()

        def gather(ix, t, buf, sem):
            pltpu.async_copy(gbf_hbm.at[ix.at[t, 0]], buf, sem)

        def wait_rows(buf, sem):
            pltpu.make_async_copy(gbf_hbm.at[ixa.at[0, 0]], buf, sem).wait()

        def scatter(ix, t, buf):
            pltpu.sync_copy(buf, acc_sh.at[ix.at[t, 1]], add=True)

        def block(ix, nxt_ix, nxt_sem, has_next):
            def pair(ti, carry):
                t = 2 * ti
                wait_rows(rows_a, sem_a)
                gather(ix, t + 1, rows_b, sem_b)
                scatter(ix, t, rows_a)
                wait_rows(rows_b, sem_b)
                gather(ix, t + 2, rows_a, sem_a)
                scatter(ix, t + 1, rows_b)
                return carry

            lax.fori_loop(0, GB // 2 - 1, pair, 0)
            t = GB - 2
            wait_rows(rows_a, sem_a)
            gather(ix, t + 1, rows_b, sem_b)
            scatter(ix, t, rows_a)
            wait_rows(rows_b, sem_b)

            @pl.when(has_next)
            def _():
                wait_stage(nxt_ix, nxt_sem)
                gather(nxt_ix, 0, rows_a, sem_a)

            scatter(ix, t + 1, rows_b)

        stage(0, ixa, sem_ia).wait()
        stage(1, ixb, sem_ib)
        gather(ixa, 0, rows_a, sem_a)

        def outer2(bi, carry):
            b0 = 2 * bi
            block(ixa, ixb, sem_ib, b0 + 1 < nblk)

            @pl.when(b0 + 2 < nblk)
            def _():
                stage(b0 + 2, ixa, sem_ia)

            block(ixb, ixa, sem_ia, b0 + 2 < nblk)

            @pl.when(b0 + 3 < nblk)
            def _():
                stage(b0 + 3, ixb, sem_ib)

            return carry

        lax.fori_loop(0, nblk // 2, outer2, 0)
        plsc.subcore_barrier()
        pltpu.sync_copy(acc_sh.at[pl.ds(s * RPT, RPT)],
                        out_hbm.at[pl.ds(c * NPAD + s * RPT, RPT)])

    return scat


def _matmul_scale(x, W, d0, d1, NPAD):
    N, Din = x.shape
    Dout = W.shape[1]
    BN = 1000

    def body(x_ref, w_ref, d0_ref, d1_ref, gbf_ref, dinv_ref):
        dinv = lax.rsqrt(d0_ref[...] + d1_ref[...] + 1.0)
        h = jnp.dot(x_ref[...], w_ref[...],
                    preferred_element_type=jnp.float32)
        gbf_ref[...] = (h * dinv).astype(jnp.bfloat16)
        dinv_ref[...] = dinv

    return pl.pallas_call(
        body,
        grid=(N // BN,),
        in_specs=[
            pl.BlockSpec((BN, Din), lambda i: (i, 0)),
            pl.BlockSpec((Din, Dout), lambda i: (0, 0)),
            pl.BlockSpec((BN, 1), lambda i: (i, 0)),
            pl.BlockSpec((BN, 1), lambda i: (i, 0)),
        ],
        out_specs=[
            pl.BlockSpec((BN, Dout), lambda i: (i, 0)),
            pl.BlockSpec((BN, 1), lambda i: (i, 0)),
        ],
        out_shape=[
            jax.ShapeDtypeStruct((NPAD, Dout), jnp.bfloat16),
            jax.ShapeDtypeStruct((N, 1), jnp.float32),
        ],
    )(x, W, d0, d1)


def _final(acc, g, dinv, b2d, N):
    D = g.shape[1]
    BN = 1000

    def body(a_ref, g_ref, dinv_ref, b_ref, o_ref):
        a = a_ref[...].astype(jnp.float32)
        o_ref[...] = (dinv_ref[...]
                      * (a[0] + a[1] - g_ref[...].astype(jnp.float32))
                      + b_ref[...])

    return pl.pallas_call(
        body,
        grid=(N // BN,),
        in_specs=[
            pl.BlockSpec((2, BN, D), lambda i: (0, i, 0)),
            pl.BlockSpec((BN, D), lambda i: (i, 0)),
            pl.BlockSpec((BN, 1), lambda i: (i, 0)),
            pl.BlockSpec((1, D), lambda i: (0, 0)),
        ],
        out_specs=pl.BlockSpec((BN, D), lambda i: (i, 0)),
        out_shape=jax.ShapeDtypeStruct((N, D), jnp.float32),
    )(acc, g, dinv, b2d)


def kernel(x, edge_index, t_embed, W, b):
    N, Din = x.shape
    Dout = W.shape[1]
    E = edge_index.shape[1]
    src = edge_index[0]
    dst = edge_index[1]

    NPAD = 10240
    KH = 80
    K = 125
    GB = 8

    nch = E // NW // K
    edg = jnp.stack(
        [src.reshape(NW, nch, K), dst.reshape(NW, nch, K)], axis=2)
    dst3dh = dst.reshape(NW, E // NW // KH, KH)

    degp = _make_hist(E, NPAD, KH)(dst3dh)
    d0 = degp[:N].reshape(N, 1)
    d1 = degp[NPAD:NPAD + N].reshape(N, 1)

    gbf, dinv = _matmul_scale(x, W, d0, d1, NPAD)

    acc = _make_scatter(NPAD, Dout, E, K, GB)(gbf, edg)
    acc = acc.reshape(NC, NPAD, Dout)

    out = _final(acc, gbf, dinv, b.reshape(1, Dout), N)
    return (out, edge_index, t_embed)


# speedup vs baseline: 1.1176x; 1.0085x over previous
import functools

import jax
import jax.numpy as jnp
from jax import lax
from jax.experimental import pallas as pl
from jax.experimental.pallas import tpu as pltpu
from jax.experimental.pallas import tpu_sc as plsc

NC = 2
NS = 16
NW = NC * NS


def _sc_mesh():
    return plsc.VectorSubcoreMesh(
        core_axis_name="c", subcore_axis_name="s",
        num_cores=NC, num_subcores=NS)


def _make_hist(E, MDEG, K):
    EPW = E // NW
    nch = EPW // K
    RPT = MDEG // NS

    KUP = (K + 15) // 16 * 16

    @functools.partial(
        pl.kernel,
        out_type=jax.ShapeDtypeStruct((NC * MDEG,), jnp.float32),
        mesh=_sc_mesh(),
        scratch_types=[
            pltpu.VMEM_SHARED((MDEG,), jnp.float32),
            pltpu.VMEM((nch, K), jnp.int32),
            pltpu.VMEM((KUP,), jnp.float32),
            pltpu.VMEM((RPT,), jnp.float32),
        ],
    )
    def hist(dst_hbm, out_hbm, deg_sh, dste, ones_v, zbuf):
        c = lax.axis_index("c")
        s = lax.axis_index("s")
        w = c * NS + s
        for i in range(RPT // 16):
            zbuf[pl.ds(i * 16, 16)] = jnp.zeros((16,), jnp.float32)
        for i in range(KUP // 16):
            ones_v[pl.ds(i * 16, 16)] = jnp.ones((16,), jnp.float32)
        pltpu.sync_copy(zbuf, deg_sh.at[pl.ds(s * RPT, RPT)])
        pltpu.sync_copy(dst_hbm.at[w], dste)
        plsc.subcore_barrier()

        def body(j, carry):
            pltpu.sync_copy(ones_v.at[pl.ds(0, K)],
                            deg_sh.at[dste.at[j]], add=True)
            return carry

        lax.fori_loop(0, nch, body, 0)
        plsc.subcore_barrier()
        pltpu.sync_copy(deg_sh.at[pl.ds(s * RPT, RPT)],
                        out_hbm.at[pl.ds(c * MDEG + s * RPT, RPT)])

    return hist


def _make_scatter(NPAD, D, E, K, GB):
    EPW = E // NW
    nch = EPW // K
    nblk = nch // GB
    RPT = NPAD // NS
    assert nch % GB == 0 and nblk % 2 == 0 and GB % 2 == 0

    @functools.partial(
        pl.kernel,
        out_type=jax.ShapeDtypeStruct((NC * NPAD, D), jnp.bfloat16),
        mesh=_sc_mesh(),
        compiler_params=pltpu.CompilerParams(use_tc_tiling_on_sc=False),
        scratch_types=[
            pltpu.VMEM_SHARED((NPAD, D), jnp.bfloat16),
            pltpu.VMEM((GB, 2, K), jnp.int32),
            pltpu.VMEM((GB, 2, K), jnp.int32),
            pltpu.VMEM((K, D), jnp.bfloat16),
            pltpu.VMEM((K, D), jnp.bfloat16),
            pltpu.SemaphoreType.DMA,
            pltpu.SemaphoreType.DMA,
            pltpu.SemaphoreType.DMA,
            pltpu.SemaphoreType.DMA,
        ],
    )
    def scat(gbf_hbm, edg_hbm, out_hbm,
             acc_sh, ixa, ixb, rows_a, rows_b, sem_a, sem_b, sem_ia, sem_ib):
        c = lax.axis_index("c")
        s = lax.axis_index("s")
        w = c * NS + s
        pltpu.sync_copy(gbf_hbm.at[pl.ds(s * RPT, RPT)],
                        acc_sh.at[pl.ds(s * RPT, RPT)])
        plsc.subcore_barrier()

        def stage(b, buf, sem):
            return pltpu.async_copy(
                edg_hbm.at[w, pl.ds(b * GB, GB)], buf, sem)

        def wait_stage(buf, sem):
            pltpu.make_async_copy(edg_hbm.at[w, pl.ds(0, GB)], buf, sem).wait()

        def gather(ix, t, buf, sem):
            pltpu.async_copy(gbf_hbm.at[ix.at[t, 0]], buf, sem)

        def wait_rows(buf, sem):
            pltpu.make_async_copy(gbf_hbm.at[ixa.at[0, 0]], buf, sem).wait()

        def scatter(ix, t, buf):
            pltpu.sync_copy(buf, acc_sh.at[ix.at[t, 1]], add=True)

        def block(ix, nxt_ix, nxt_sem, has_next):
            def pair(ti, carry):
                t = 2 * ti
                wait_rows(rows_a, sem_a)
                gather(ix, t + 1, rows_b, sem_b)
                scatter(ix, t, rows_a)
                wait_rows(rows_b, sem_b)
                gather(ix, t + 2, rows_a, sem_a)
                scatter(ix, t + 1, rows_b)
                return carry

            lax.fori_loop(0, GB // 2 - 1, pair, 0)
            t = GB - 2
            wait_rows(rows_a, sem_a)
            gather(ix, t + 1, rows_b, sem_b)
            scatter(ix, t, rows_a)
            wait_rows(rows_b, sem_b)

            @pl.when(has_next)
            def _():
                wait_stage(nxt_ix, nxt_sem)
                gather(nxt_ix, 0, rows_a, sem_a)

            scatter(ix, t + 1, rows_b)

        stage(0, ixa, sem_ia).wait()
        stage(1, ixb, sem_ib)
        gather(ixa, 0, rows_a, sem_a)

        def outer2(bi, carry):
            b0 = 2 * bi
            block(ixa, ixb, sem_ib, b0 + 1 < nblk)

            @pl.when(b0 + 2 < nblk)
            def _():
                stage(b0 + 2, ixa, sem_ia)

            block(ixb, ixa, sem_ia, b0 + 2 < nblk)

            @pl.when(b0 + 3 < nblk)
            def _():
                stage(b0 + 3, ixb, sem_ib)

            return carry

        lax.fori_loop(0, nblk // 2, outer2, 0)
        plsc.subcore_barrier()
        pltpu.sync_copy(acc_sh.at[pl.ds(s * RPT, RPT)],
                        out_hbm.at[pl.ds(c * NPAD + s * RPT, RPT)])

    return scat


def _matmul_scale(x, W, d0, d1, NPAD):
    N, Din = x.shape
    Dout = W.shape[1]
    BN = 1000

    def body(x_ref, w_ref, d0_ref, d1_ref, gbf_ref, dinv_ref):
        dinv = lax.rsqrt(d0_ref[...] + d1_ref[...] + 1.0)
        h = jnp.dot(x_ref[...], w_ref[...],
                    preferred_element_type=jnp.float32)
        gbf_ref[...] = (h * dinv).astype(jnp.bfloat16)
        dinv_ref[...] = dinv

    return pl.pallas_call(
        body,
        grid=(N // BN,),
        in_specs=[
            pl.BlockSpec((BN, Din), lambda i: (i, 0)),
            pl.BlockSpec((Din, Dout), lambda i: (0, 0)),
            pl.BlockSpec((BN, 1), lambda i: (i, 0)),
            pl.BlockSpec((BN, 1), lambda i: (i, 0)),
        ],
        out_specs=[
            pl.BlockSpec((BN, Dout), lambda i: (i, 0)),
            pl.BlockSpec((BN, 1), lambda i: (i, 0)),
        ],
        out_shape=[
            jax.ShapeDtypeStruct((NPAD, Dout), jnp.bfloat16),
            jax.ShapeDtypeStruct((N, 1), jnp.float32),
        ],
    )(x, W, d0, d1)


def _final(acc, g, dinv, b2d, N):
    D = g.shape[1]
    BN = 1000

    def body(a_ref, g_ref, dinv_ref, b_ref, o_ref):
        a = a_ref[...].astype(jnp.float32)
        o_ref[...] = (dinv_ref[...]
                      * (a[0] + a[1] - g_ref[...].astype(jnp.float32))
                      + b_ref[...])

    return pl.pallas_call(
        body,
        grid=(N // BN,),
        in_specs=[
            pl.BlockSpec((2, BN, D), lambda i: (0, i, 0)),
            pl.BlockSpec((BN, D), lambda i: (i, 0)),
            pl.BlockSpec((BN, 1), lambda i: (i, 0)),
            pl.BlockSpec((1, D), lambda i: (0, 0)),
        ],
        out_specs=pl.BlockSpec((BN, D), lambda i: (i, 0)),
        out_shape=jax.ShapeDtypeStruct((N, D), jnp.float32),
    )(acc, g, dinv, b2d)


def kernel(x, edge_index, t_embed, W, b):
    N, Din = x.shape
    Dout = W.shape[1]
    E = edge_index.shape[1]
    src = edge_index[0]
    dst = edge_index[1]

    NPAD = 10240
    KH = 125
    K = 125
    GB = 8

    nch = E // NW // K
    edg = jnp.stack(
        [src.reshape(NW, nch, K), dst.reshape(NW, nch, K)], axis=2)
    dst3dh = dst.reshape(NW, E // NW // KH, KH)

    degp = _make_hist(E, NPAD, KH)(dst3dh)
    del dst3dh
    d0 = degp[:N].reshape(N, 1)
    d1 = degp[NPAD:NPAD + N].reshape(N, 1)

    gbf, dinv = _matmul_scale(x, W, d0, d1, NPAD)

    acc = _make_scatter(NPAD, Dout, E, K, GB)(gbf, edg)
    acc = acc.reshape(NC, NPAD, Dout)

    out = _final(acc, gbf, dinv, b.reshape(1, Dout), N)
    return (out, edge_index, t_embed)


# speedup vs baseline: 1.1397x; 1.0198x over previous
import functools

import jax
import jax.numpy as jnp
from jax import lax
from jax.experimental import pallas as pl
from jax.experimental.pallas import tpu as pltpu
from jax.experimental.pallas import tpu_sc as plsc

NC = 2
NS = 16
NW = NC * NS


def _sc_mesh():
    return plsc.VectorSubcoreMesh(
        core_axis_name="c", subcore_axis_name="s",
        num_cores=NC, num_subcores=NS)


def _make_hist(E, MDEG, K):
    EPW = E // NW
    nch = EPW // K
    RPT = MDEG // NS

    KUP = (K + 15) // 16 * 16

    @functools.partial(
        pl.kernel,
        out_type=jax.ShapeDtypeStruct((NC * MDEG,), jnp.float32),
        mesh=_sc_mesh(),
        scratch_types=[
            pltpu.VMEM_SHARED((MDEG,), jnp.float32),
            pltpu.VMEM((nch, K), jnp.int32),
            pltpu.VMEM((KUP,), jnp.float32),
            pltpu.VMEM((RPT,), jnp.float32),
        ],
    )
    def hist(dst_hbm, out_hbm, deg_sh, dste, ones_v, zbuf):
        c = lax.axis_index("c")
        s = lax.axis_index("s")
        w = c * NS + s
        for i in range(RPT // 16):
            zbuf[pl.ds(i * 16, 16)] = jnp.zeros((16,), jnp.float32)
        for i in range(KUP // 16):
            ones_v[pl.ds(i * 16, 16)] = jnp.ones((16,), jnp.float32)
        pltpu.sync_copy(zbuf, deg_sh.at[pl.ds(s * RPT, RPT)])
        pltpu.sync_copy(dst_hbm.at[w], dste)
        plsc.subcore_barrier()

        def body(j, carry):
            pltpu.sync_copy(ones_v.at[pl.ds(0, K)],
                            deg_sh.at[dste.at[j]], add=True)
            return carry

        lax.fori_loop(0, nch, body, 0)
        plsc.subcore_barrier()
        pltpu.sync_copy(deg_sh.at[pl.ds(s * RPT, RPT)],
                        out_hbm.at[pl.ds(c * MDEG + s * RPT, RPT)])

    return hist


def _make_scatter(NPAD, D, E, K, GB):
    EPW = E // NW
    nch = EPW // K
    nblk = nch // GB
    RPT = NPAD // NS
    assert nch % GB == 0 and nblk % 2 == 0 and GB % 2 == 0

    @functools.partial(
        pl.kernel,
        out_type=jax.ShapeDtypeStruct((NC * NPAD, D), jnp.bfloat16),
        mesh=_sc_mesh(),
        compiler_params=pltpu.CompilerParams(use_tc_tiling_on_sc=False),
        scratch_types=[
            pltpu.VMEM_SHARED((NPAD, D), jnp.bfloat16),
            pltpu.VMEM((GB, 2, K), jnp.int32),
            pltpu.VMEM((GB, 2, K), jnp.int32),
            pltpu.VMEM((K, D), jnp.bfloat16),
            pltpu.VMEM((K, D), jnp.bfloat16),
            pltpu.SemaphoreType.DMA,
            pltpu.SemaphoreType.DMA,
            pltpu.SemaphoreType.DMA,
            pltpu.SemaphoreType.DMA,
            pltpu.SemaphoreType.DMA,
            pltpu.SemaphoreType.DMA,
        ],
    )
    def scat(gbf_hbm, edg_hbm, out_hbm,
             acc_sh, ixa, ixb, rows_a, rows_b,
             sem_a, sem_b, sem_sa, sem_sb, sem_ia, sem_ib):
        c = lax.axis_index("c")
        s = lax.axis_index("s")
        w = c * NS + s
        pltpu.sync_copy(gbf_hbm.at[pl.ds(s * RPT, RPT)],
                        acc_sh.at[pl.ds(s * RPT, RPT)])
        plsc.subcore_barrier()

        def stage(b, buf, sem):
            return pltpu.async_copy(
                edg_hbm.at[w, pl.ds(b * GB, GB)], buf, sem)

        def wait_stage(buf, sem):
            pltpu.make_async_copy(edg_hbm.at[w, pl.ds(0, GB)], buf, sem).wait()

        def gather(ix, t, buf, sem):
            pltpu.async_copy(gbf_hbm.at[ix.at[t, 0]], buf, sem)

        def wait_rows(buf, sem):
            pltpu.make_async_copy(gbf_hbm.at[ixa.at[0, 0]], buf, sem).wait()

        def scatter(ix, t, buf, sem):
            pltpu.async_copy(buf, acc_sh.at[ix.at[t, 1]], sem, add=True)

        def wait_scat(buf, sem):
            pltpu.make_async_copy(buf, acc_sh.at[ixa.at[0, 1]], sem).wait()

        def block(ix, nxt_ix, nxt_sem, has_next):
            def pair(ti, carry):
                t = 2 * ti
                wait_rows(rows_a, sem_a)
                scatter(ix, t, rows_a, sem_sa)
                wait_rows(rows_b, sem_b)
                scatter(ix, t + 1, rows_b, sem_sb)
                wait_scat(rows_a, sem_sa)
                gather(ix, t + 2, rows_a, sem_a)
                wait_scat(rows_b, sem_sb)
                gather(ix, t + 3, rows_b, sem_b)
                return carry

            lax.fori_loop(0, GB // 2 - 1, pair, 0)
            t = GB - 2
            wait_rows(rows_a, sem_a)
            scatter(ix, t, rows_a, sem_sa)
            wait_rows(rows_b, sem_b)
            scatter(ix, t + 1, rows_b, sem_sb)

            @pl.when(has_next)
            def _():
                wait_stage(nxt_ix, nxt_sem)

            wait_scat(rows_a, sem_sa)

            @pl.when(has_next)
            def _():
                gather(nxt_ix, 0, rows_a, sem_a)

            wait_scat(rows_b, sem_sb)

            @pl.when(has_next)
            def _():
                gather(nxt_ix, 1, rows_b, sem_b)

        stage(0, ixa, sem_ia).wait()
        stage(1, ixb, sem_ib)
        gather(ixa, 0, rows_a, sem_a)
        gather(ixa, 1, rows_b, sem_b)

        def outer2(bi, carry):
            b0 = 2 * bi
            block(ixa, ixb, sem_ib, b0 + 1 < nblk)

            @pl.when(b0 + 2 < nblk)
            def _():
                stage(b0 + 2, ixa, sem_ia)

            block(ixb, ixa, sem_ia, b0 + 2 < nblk)

            @pl.when(b0 + 3 < nblk)
            def _():
                stage(b0 + 3, ixb, sem_ib)

            return carry

        lax.fori_loop(0, nblk // 2, outer2, 0)
        plsc.subcore_barrier()
        pltpu.sync_copy(acc_sh.at[pl.ds(s * RPT, RPT)],
                        out_hbm.at[pl.ds(c * NPAD + s * RPT, RPT)])

    return scat


def _matmul_scale(x, W, d0, d1, NPAD):
    N, Din = x.shape
    Dout = W.shape[1]
    BN = 1000

    def body(x_ref, w_ref, d0_ref, d1_ref, gbf_ref, dinv_ref):
        dinv = lax.rsqrt(d0_ref[...] + d1_ref[...] + 1.0)
        h = jnp.dot(x_ref[...], w_ref[...],
                    preferred_element_type=jnp.float32)
        gbf_ref[...] = (h * dinv).astype(jnp.bfloat16)
        dinv_ref[...] = dinv

    return pl.pallas_call(
        body,
        grid=(N // BN,),
        in_specs=[
            pl.BlockSpec((BN, Din), lambda i: (i, 0)),
            pl.BlockSpec((Din, Dout), lambda i: (0, 0)),
            pl.BlockSpec((BN, 1), lambda i: (i, 0)),
            pl.BlockSpec((BN, 1), lambda i: (i, 0)),
        ],
        out_specs=[
            pl.BlockSpec((BN, Dout), lambda i: (i, 0)),
            pl.BlockSpec((BN, 1), lambda i: (i, 0)),
        ],
        out_shape=[
            jax.ShapeDtypeStruct((NPAD, Dout), jnp.bfloat16),
            jax.ShapeDtypeStruct((N, 1), jnp.float32),
        ],
    )(x, W, d0, d1)


def _final(acc, g, dinv, b2d, N):
    D = g.shape[1]
    BN = 1000

    def body(a_ref, g_ref, dinv_ref, b_ref, o_ref):
        a = a_ref[...].astype(jnp.float32)
        o_ref[...] = (dinv_ref[...]
                      * (a[0] + a[1] - g_ref[...].astype(jnp.float32))
                      + b_ref[...])

    return pl.pallas_call(
        body,
        grid=(N // BN,),
        in_specs=[
            pl.BlockSpec((2, BN, D), lambda i: (0, i, 0)),
            pl.BlockSpec((BN, D), lambda i: (i, 0)),
            pl.BlockSpec((BN, 1), lambda i: (i, 0)),
            pl.BlockSpec((1, D), lambda i: (0, 0)),
        ],
        out_specs=pl.BlockSpec((BN, D), lambda i: (i, 0)),
        out_shape=jax.ShapeDtypeStruct((N, D), jnp.float32),
    )(acc, g, dinv, b2d)


def kernel(x, edge_index, t_embed, W, b):
    N, Din = x.shape
    Dout = W.shape[1]
    E = edge_index.shape[1]
    src = edge_index[0]
    dst = edge_index[1]

    NPAD = 10240
    KH = 125
    K = 125
    GB = 8

    nch = E // NW // K
    edg = jnp.stack(
        [src.reshape(NW, nch, K), dst.reshape(NW, nch, K)], axis=2)
    dst3dh = dst.reshape(NW, E // NW // KH, KH)

    degp = _make_hist(E, NPAD, KH)(dst3dh)
    del dst3dh
    d0 = degp[:N].reshape(N, 1)
    d1 = degp[NPAD:NPAD + N].reshape(N, 1)

    gbf, dinv = _matmul_scale(x, W, d0, d1, NPAD)

    acc = _make_scatter(NPAD, Dout, E, K, GB)(gbf, edg)
    acc = acc.reshape(NC, NPAD, Dout)

    out = _final(acc, gbf, dinv, b.reshape(1, Dout), N)
    return (out, edge_index, t_embed)


# speedup vs baseline: 1.2679x; 1.1125x over previous
import functools

import jax
import jax.numpy as jnp
from jax import lax
from jax.experimental import pallas as pl
from jax.experimental.pallas import tpu as pltpu
from jax.experimental.pallas import tpu_sc as plsc

NC = 2
NS = 16
NW = NC * NS


def _sc_mesh():
    return plsc.VectorSubcoreMesh(
        core_axis_name="c", subcore_axis_name="s",
        num_cores=NC, num_subcores=NS)


def _make_hist(E, MDEG, K):
    EPW = E // NW
    nch = EPW // K
    RPT = MDEG // NS

    KUP = (K + 15) // 16 * 16

    @functools.partial(
        pl.kernel,
        out_type=jax.ShapeDtypeStruct((NC * MDEG,), jnp.float32),
        mesh=_sc_mesh(),
        scratch_types=[
            pltpu.VMEM_SHARED((MDEG,), jnp.float32),
            pltpu.VMEM((nch, K), jnp.int32),
            pltpu.VMEM((KUP,), jnp.float32),
            pltpu.VMEM((RPT,), jnp.float32),
        ],
    )
    def hist(dst_hbm, out_hbm, deg_sh, dste, ones_v, zbuf):
        c = lax.axis_index("c")
        s = lax.axis_index("s")
        w = c * NS + s
        for i in range(RPT // 16):
            zbuf[pl.ds(i * 16, 16)] = jnp.zeros((16,), jnp.float32)
        for i in range(KUP // 16):
            ones_v[pl.ds(i * 16, 16)] = jnp.ones((16,), jnp.float32)
        pltpu.sync_copy(zbuf, deg_sh.at[pl.ds(s * RPT, RPT)])
        pltpu.sync_copy(dst_hbm.at[w], dste)
        plsc.subcore_barrier()

        def body(j, carry):
            pltpu.sync_copy(ones_v.at[pl.ds(0, K)],
                            deg_sh.at[dste.at[j]], add=True)
            return carry

        lax.fori_loop(0, nch, body, 0)
        plsc.subcore_barrier()
        pltpu.sync_copy(deg_sh.at[pl.ds(s * RPT, RPT)],
                        out_hbm.at[pl.ds(c * MDEG + s * RPT, RPT)])

    return hist


def _make_scatter(NPAD, D, E, K, GB):
    EPW = E // NW
    nch = EPW // K
    nblk = nch // GB
    RPT = NPAD // NS
    assert nch % GB == 0 and nblk % 2 == 0 and GB % 2 == 0

    @functools.partial(
        pl.kernel,
        out_type=jax.ShapeDtypeStruct((NC * NPAD, D), jnp.bfloat16),
        mesh=_sc_mesh(),
        compiler_params=pltpu.CompilerParams(use_tc_tiling_on_sc=False),
        scratch_types=[
            pltpu.VMEM_SHARED((NPAD, D), jnp.bfloat16),
            pltpu.VMEM((GB, 2, K), jnp.int32),
            pltpu.VMEM((GB, 2, K), jnp.int32),
            pltpu.VMEM((K, D), jnp.bfloat16),
            pltpu.VMEM((K, D), jnp.bfloat16),
            pltpu.VMEM((K, D), jnp.bfloat16),
            pltpu.VMEM((K, D), jnp.bfloat16),
            pltpu.SemaphoreType.DMA,
            pltpu.SemaphoreType.DMA,
            pltpu.SemaphoreType.DMA,
            pltpu.SemaphoreType.DMA,
            pltpu.SemaphoreType.DMA,
            pltpu.SemaphoreType.DMA,
            pltpu.SemaphoreType.DMA,
            pltpu.SemaphoreType.DMA,
            pltpu.SemaphoreType.DMA,
            pltpu.SemaphoreType.DMA,
        ],
    )
    def scat(gbf_hbm, edg_hbm, out_hbm,
             acc_sh, ixa, ixb, r0, r1, r2, r3,
             g0, g1, g2, g3, s0, s1, s2, s3, sem_ia, sem_ib):
        rows = (r0, r1, r2, r3)
        gsem = (g0, g1, g2, g3)
        ssem = (s0, s1, s2, s3)
        c = lax.axis_index("c")
        s = lax.axis_index("s")
        w = c * NS + s
        pltpu.sync_copy(gbf_hbm.at[pl.ds(s * RPT, RPT)],
                        acc_sh.at[pl.ds(s * RPT, RPT)])
        plsc.subcore_barrier()

        def stage(b, buf, sem):
            return pltpu.async_copy(
                edg_hbm.at[w, pl.ds(b * GB, GB)], buf, sem)

        def wait_stage(buf, sem):
            pltpu.make_async_copy(edg_hbm.at[w, pl.ds(0, GB)], buf, sem).wait()

        def gather(ix, t, buf, sem):
            pltpu.async_copy(gbf_hbm.at[ix.at[t, 0]], buf, sem)

        def wait_rows(buf, sem):
            pltpu.make_async_copy(gbf_hbm.at[ixa.at[0, 0]], buf, sem).wait()

        def scatter(ix, t, buf, sem):
            pltpu.async_copy(buf, acc_sh.at[ix.at[t, 1]], sem, add=True)

        def wait_scat(buf, sem):
            pltpu.make_async_copy(buf, acc_sh.at[ixa.at[0, 1]], sem).wait()

        def step(jglob, ix, t, gx, gt, has_g, q):
            q2 = (q + 2) % 4
            wait_rows(rows[q], gsem[q])
            scatter(ix, t, rows[q], ssem[q])

            @pl.when(jglob >= 2)
            def _():
                wait_scat(rows[q2], ssem[q2])

            @pl.when(has_g)
            def _():
                gather(gx, gt, rows[q2], gsem[q2])

        def block(b, ix, nxt_ix, nxt_sem, has_next):
            def quad(ti, carry):
                t = 4 * ti
                for k in range(4):
                    step(b * GB + t + k, ix, t + k, ix, t + k + 2,
                         jnp.bool_(True), k)
                return carry

            lax.fori_loop(0, GB // 4 - 1, quad, 0)
            t = GB - 4
            step(b * GB + t, ix, t, ix, t + 2, jnp.bool_(True), 0)
            step(b * GB + t + 1, ix, t + 1, ix, t + 3, jnp.bool_(True), 1)

            @pl.when(has_next)
            def _():
                wait_stage(nxt_ix, nxt_sem)

            step(b * GB + t + 2, ix, t + 2, nxt_ix, 0, has_next, 2)
            step(b * GB + t + 3, ix, t + 3, nxt_ix, 1, has_next, 3)

        stage(0, ixa, sem_ia).wait()
        stage(1, ixb, sem_ib)
        gather(ixa, 0, rows[0], gsem[0])
        gather(ixa, 1, rows[1], gsem[1])

        def outer2(bi, carry):
            b0 = 2 * bi
            block(b0, ixa, ixb, sem_ib, b0 + 1 < nblk)

            @pl.when(b0 + 2 < nblk)
            def _():
                stage(b0 + 2, ixa, sem_ia)

            block(b0 + 1, ixb, ixa, sem_ia, b0 + 2 < nblk)

            @pl.when(b0 + 3 < nblk)
            def _():
                stage(b0 + 3, ixb, sem_ib)

            return carry

        lax.fori_loop(0, nblk // 2, outer2, 0)
        wait_scat(rows[(nch - 2) % 4], ssem[(nch - 2) % 4])
        wait_scat(rows[(nch - 1) % 4], ssem[(nch - 1) % 4])
        plsc.subcore_barrier()
        pltpu.sync_copy(acc_sh.at[pl.ds(s * RPT, RPT)],
                        out_hbm.at[pl.ds(c * NPAD + s * RPT, RPT)])

    return scat


def _matmul_scale(x, W, d0, d1, NPAD):
    N, Din = x.shape
    Dout = W.shape[1]
    BN = 1000

    def body(x_ref, w_ref, d0_ref, d1_ref, gbf_ref, dinv_ref):
        dinv = lax.rsqrt(d0_ref[...] + d1_ref[...] + 1.0)
        h = jnp.dot(x_ref[...], w_ref[...],
                    preferred_element_type=jnp.float32)
        gbf_ref[...] = (h * dinv).astype(jnp.bfloat16)
        dinv_ref[...] = dinv

    return pl.pallas_call(
        body,
        grid=(N // BN,),
        in_specs=[
            pl.BlockSpec((BN, Din), lambda i: (i, 0)),
            pl.BlockSpec((Din, Dout), lambda i: (0, 0)),
            pl.BlockSpec((BN, 1), lambda i: (i, 0)),
            pl.BlockSpec((BN, 1), lambda i: (i, 0)),
        ],
        out_specs=[
            pl.BlockSpec((BN, Dout), lambda i: (i, 0)),
            pl.BlockSpec((BN, 1), lambda i: (i, 0)),
        ],
        out_shape=[
            jax.ShapeDtypeStruct((NPAD, Dout), jnp.bfloat16),
            jax.ShapeDtypeStruct((N, 1), jnp.float32),
        ],
    )(x, W, d0, d1)


def _final(acc, g, dinv, b2d, N):
    D = g.shape[1]
    BN = 1000

    def body(a_ref, g_ref, dinv_ref, b_ref, o_ref):
        a = a_ref[...].astype(jnp.float32)
        o_ref[...] = (dinv_ref[...]
                      * (a[0] + a[1] - g_ref[...].astype(jnp.float32))
                      + b_ref[...])

    return pl.pallas_call(
        body,
        grid=(N // BN,),
        in_specs=[
            pl.BlockSpec((2, BN, D), lambda i: (0, i, 0)),
            pl.BlockSpec((BN, D), lambda i: (i, 0)),
            pl.BlockSpec((BN, 1), lambda i: (i, 0)),
            pl.BlockSpec((1, D), lambda i: (0, 0)),
        ],
        out_specs=pl.BlockSpec((BN, D), lambda i: (i, 0)),
        out_shape=jax.ShapeDtypeStruct((N, D), jnp.float32),
    )(acc, g, dinv, b2d)


def kernel(x, edge_index, t_embed, W, b):
    N, Din = x.shape
    Dout = W.shape[1]
    E = edge_index.shape[1]
    src = edge_index[0]
    dst = edge_index[1]

    NPAD = 10240
    KH = 125
    K = 125
    GB = 8

    nch = E // NW // K
    edg = jnp.stack(
        [src.reshape(NW, nch, K), dst.reshape(NW, nch, K)], axis=2)
    dst3dh = dst.reshape(NW, E // NW // KH, KH)

    degp = _make_hist(E, NPAD, KH)(dst3dh)
    del dst3dh
    d0 = degp[:N].reshape(N, 1)
    d1 = degp[NPAD:NPAD + N].reshape(N, 1)

    gbf, dinv = _matmul_scale(x, W, d0, d1, NPAD)

    acc = _make_scatter(NPAD, Dout, E, K, GB)(gbf, edg)
    acc = acc.reshape(NC, NPAD, Dout)

    out = _final(acc, gbf, dinv, b.reshape(1, Dout), N)
    return (out, edge_index, t_embed)


# speedup vs baseline: 1.3168x; 1.0385x over previous
import functools

import jax
import jax.numpy as jnp
from jax import lax
from jax.experimental import pallas as pl
from jax.experimental.pallas import tpu as pltpu
from jax.experimental.pallas import tpu_sc as plsc

NC = 2
NS = 16
NW = NC * NS


def _sc_mesh():
    return plsc.VectorSubcoreMesh(
        core_axis_name="c", subcore_axis_name="s",
        num_cores=NC, num_subcores=NS)


def _make_hist(E, MDEG, K):
    EPW = E // NW
    nch = EPW // K
    RPT = MDEG // NS

    KUP = (K + 15) // 16 * 16

    @functools.partial(
        pl.kernel,
        out_type=jax.ShapeDtypeStruct((NC * MDEG,), jnp.float32),
        mesh=_sc_mesh(),
        scratch_types=[
            pltpu.VMEM_SHARED((MDEG,), jnp.float32),
            pltpu.VMEM((nch, K), jnp.int32),
            pltpu.VMEM((KUP,), jnp.float32),
            pltpu.VMEM((RPT,), jnp.float32),
        ],
    )
    def hist(dst_hbm, out_hbm, deg_sh, dste, ones_v, zbuf):
        c = lax.axis_index("c")
        s = lax.axis_index("s")
        w = c * NS + s
        for i in range(RPT // 16):
            zbuf[pl.ds(i * 16, 16)] = jnp.zeros((16,), jnp.float32)
        for i in range(KUP // 16):
            ones_v[pl.ds(i * 16, 16)] = jnp.ones((16,), jnp.float32)
        pltpu.sync_copy(zbuf, deg_sh.at[pl.ds(s * RPT, RPT)])
        pltpu.sync_copy(dst_hbm.at[w], dste)
        plsc.subcore_barrier()

        def body(j, carry):
            pltpu.sync_copy(ones_v.at[pl.ds(0, K)],
                            deg_sh.at[dste.at[j]], add=True)
            return carry

        lax.fori_loop(0, nch, body, 0)
        plsc.subcore_barrier()
        pltpu.sync_copy(deg_sh.at[pl.ds(s * RPT, RPT)],
                        out_hbm.at[pl.ds(c * MDEG + s * RPT, RPT)])

    return hist


def _make_scatter(NPAD, D, E, K, GB):
    EPW = E // NW
    nch = EPW // K
    nblk = nch // GB
    RPT = NPAD // NS
    assert nch % GB == 0 and nblk % 2 == 0 and GB % 2 == 0

    @functools.partial(
        pl.kernel,
        out_type=jax.ShapeDtypeStruct((NC * NPAD, D), jnp.bfloat16),
        mesh=_sc_mesh(),
        compiler_params=pltpu.CompilerParams(use_tc_tiling_on_sc=False),
        scratch_types=[
            pltpu.VMEM_SHARED((NPAD, D), jnp.bfloat16),
            pltpu.VMEM((GB, K), jnp.int32),
            pltpu.VMEM((GB, K), jnp.int32),
            pltpu.VMEM((nch, K), jnp.int32),
            [pltpu.VMEM((K, D), jnp.bfloat16)] * 8,
            [pltpu.SemaphoreType.DMA] * 8,
            [pltpu.SemaphoreType.DMA] * 8,
            pltpu.SemaphoreType.DMA,
            pltpu.SemaphoreType.DMA,
        ],
    )
    def scat(gbf_hbm, src_hbm, dst_hbm, out_hbm,
             acc_sh, ixa, ixb, dsta, rows, gsem, ssem, sem_ia, sem_ib):
        c = lax.axis_index("c")
        s = lax.axis_index("s")
        w = c * NS + s
        pltpu.sync_copy(dst_hbm.at[w], dsta)
        pltpu.sync_copy(gbf_hbm.at[pl.ds(s * RPT, RPT)],
                        acc_sh.at[pl.ds(s * RPT, RPT)])
        plsc.subcore_barrier()

        def stage(b, buf, sem):
            return pltpu.async_copy(
                src_hbm.at[w, pl.ds(b * GB, GB)], buf, sem)

        def wait_stage(buf, sem):
            pltpu.make_async_copy(src_hbm.at[w, pl.ds(0, GB)], buf, sem).wait()

        def gather(ix, t, buf, sem):
            pltpu.async_copy(gbf_hbm.at[ix.at[t]], buf, sem)

        def wait_rows(buf, sem):
            pltpu.make_async_copy(gbf_hbm.at[ixa.at[0]], buf, sem).wait()

        def scatter(j, buf, sem):
            pltpu.async_copy(buf, acc_sh.at[dsta.at[j]], sem, add=True)

        def wait_scat(buf, sem):
            pltpu.make_async_copy(buf, acc_sh.at[dsta.at[0]], sem).wait()

        DEPTH = 4
        NB = 2 * DEPTH

        def step(jglob, ix, t, gx, gt, has_g, q):
            q2 = (q + DEPTH) % NB
            wait_rows(rows[q], gsem[q])
            scatter(jglob, rows[q], ssem[q])

            @pl.when(jglob >= DEPTH)
            def _():
                wait_scat(rows[q2], ssem[q2])

            @pl.when(has_g)
            def _():
                gather(gx, gt, rows[q2], gsem[q2])

        def block(b, ix, nxt_ix, nxt_sem, has_next):
            for t in range(DEPTH):
                step(b * GB + t, ix, t, ix, t + DEPTH, jnp.bool_(True), t)

            @pl.when(has_next)
            def _():
                wait_stage(nxt_ix, nxt_sem)

            for t in range(DEPTH, NB):
                step(b * GB + t, ix, t, nxt_ix, t - DEPTH, has_next, t)

        stage(0, ixa, sem_ia).wait()
        stage(1, ixb, sem_ib)
        for t in range(DEPTH):
            gather(ixa, t, rows[t], gsem[t])

        def outer2(bi, carry):
            b0 = 2 * bi
            block(b0, ixa, ixb, sem_ib, b0 + 1 < nblk)

            @pl.when(b0 + 2 < nblk)
            def _():
                stage(b0 + 2, ixa, sem_ia)

            block(b0 + 1, ixb, ixa, sem_ia, b0 + 2 < nblk)

            @pl.when(b0 + 3 < nblk)
            def _():
                stage(b0 + 3, ixb, sem_ib)

            return carry

        lax.fori_loop(0, nblk // 2, outer2, 0)
        for j in range(nch - DEPTH, nch):
            wait_scat(rows[j % NB], ssem[j % NB])
        plsc.subcore_barrier()
        pltpu.sync_copy(acc_sh.at[pl.ds(s * RPT, RPT)],
                        out_hbm.at[pl.ds(c * NPAD + s * RPT, RPT)])

    return scat


def _matmul_scale(x, W, d0, d1, NPAD):
    N, Din = x.shape
    Dout = W.shape[1]
    BN = 1000

    def body(x_ref, w_ref, d0_ref, d1_ref, gbf_ref, dinv_ref):
        dinv = lax.rsqrt(d0_ref[...] + d1_ref[...] + 1.0)
        h = jnp.dot(x_ref[...], w_ref[...],
                    preferred_element_type=jnp.float32)
        gbf_ref[...] = (h * dinv).astype(jnp.bfloat16)
        dinv_ref[...] = dinv

    return pl.pallas_call(
        body,
        grid=(N // BN,),
        in_specs=[
            pl.BlockSpec((BN, Din), lambda i: (i, 0)),
            pl.BlockSpec((Din, Dout), lambda i: (0, 0)),
            pl.BlockSpec((BN, 1), lambda i: (i, 0)),
            pl.BlockSpec((BN, 1), lambda i: (i, 0)),
        ],
        out_specs=[
            pl.BlockSpec((BN, Dout), lambda i: (i, 0)),
            pl.BlockSpec((BN, 1), lambda i: (i, 0)),
        ],
        out_shape=[
            jax.ShapeDtypeStruct((NPAD, Dout), jnp.bfloat16),
            jax.ShapeDtypeStruct((N, 1), jnp.float32),
        ],
    )(x, W, d0, d1)


def _final(acc, g, dinv, b2d, N):
    D = g.shape[1]
    BN = 1000

    def body(a_ref, g_ref, dinv_ref, b_ref, o_ref):
        a = a_ref[...].astype(jnp.float32)
        o_ref[...] = (dinv_ref[...]
                      * (a[0] + a[1] - g_ref[...].astype(jnp.float32))
                      + b_ref[...])

    return pl.pallas_call(
        body,
        grid=(N // BN,),
        in_specs=[
            pl.BlockSpec((2, BN, D), lambda i: (0, i, 0)),
            pl.BlockSpec((BN, D), lambda i: (i, 0)),
            pl.BlockSpec((BN, 1), lambda i: (i, 0)),
            pl.BlockSpec((1, D), lambda i: (0, 0)),
        ],
        out_specs=pl.BlockSpec((BN, D), lambda i: (i, 0)),
        out_shape=jax.ShapeDtypeStruct((N, D), jnp.float32),
    )(acc, g, dinv, b2d)


def kernel(x, edge_index, t_embed, W, b):
    N, Din = x.shape
    Dout = W.shape[1]
    E = edge_index.shape[1]
    src = edge_index[0]
    dst = edge_index[1]

    NPAD = 10240
    KH = 125
    K = 125
    GB = 8

    nch = E // NW // K
    src3d = src.reshape(NW, nch, K)
    dst3d = dst.reshape(NW, nch, K)
    dst3dh = dst.reshape(NW, E // NW // KH, KH)

    degp = _make_hist(E, NPAD, KH)(dst3dh)
    del dst3dh
    d0 = degp[:N].reshape(N, 1)
    d1 = degp[NPAD:NPAD + N].reshape(N, 1)

    gbf, dinv = _matmul_scale(x, W, d0, d1, NPAD)

    acc = _make_scatter(NPAD, Dout, E, K, GB)(gbf, src3d, dst3d)
    acc = acc.reshape(NC, NPAD, Dout)

    out = _final(acc, gbf, dinv, b.reshape(1, Dout), N)
    return (out, edge_index, t_embed)


# speedup vs baseline: 1.3221x; 1.0040x over previous
import functools

import jax
import jax.numpy as jnp
from jax import lax
from jax.experimental import pallas as pl
from jax.experimental.pallas import tpu as pltpu
from jax.experimental.pallas import tpu_sc as plsc

NC = 2
NS = 16
NW = NC * NS


def _sc_mesh():
    return plsc.VectorSubcoreMesh(
        core_axis_name="c", subcore_axis_name="s",
        num_cores=NC, num_subcores=NS)


def _make_hist(E, MDEG, K):
    EPW = E // NW
    nch = EPW // K
    RPT = MDEG // NS

    KUP = (K + 15) // 16 * 16

    @functools.partial(
        pl.kernel,
        out_type=jax.ShapeDtypeStruct((NC * MDEG,), jnp.float32),
        mesh=_sc_mesh(),
        compiler_params=pltpu.CompilerParams(use_tc_tiling_on_sc=False),
        scratch_types=[
            pltpu.VMEM_SHARED((MDEG,), jnp.float32),
            pltpu.VMEM((nch, K), jnp.int32),
            pltpu.VMEM((KUP,), jnp.float32),
            pltpu.VMEM((RPT,), jnp.float32),
        ],
    )
    def hist(dst_hbm, out_hbm, deg_sh, dste, ones_v, zbuf):
        c = lax.axis_index("c")
        s = lax.axis_index("s")
        w = c * NS + s
        for i in range(RPT // 16):
            zbuf[pl.ds(i * 16, 16)] = jnp.zeros((16,), jnp.float32)
        for i in range(KUP // 16):
            ones_v[pl.ds(i * 16, 16)] = jnp.ones((16,), jnp.float32)
        pltpu.sync_copy(zbuf, deg_sh.at[pl.ds(s * RPT, RPT)])
        pltpu.sync_copy(dst_hbm.at[w], dste)
        plsc.subcore_barrier()

        def body(j, carry):
            pltpu.sync_copy(ones_v.at[pl.ds(0, K)],
                            deg_sh.at[dste.at[j]], add=True)
            return carry

        lax.fori_loop(0, nch, body, 0)
        plsc.subcore_barrier()
        pltpu.sync_copy(deg_sh.at[pl.ds(s * RPT, RPT)],
                        out_hbm.at[pl.ds(c * MDEG + s * RPT, RPT)])

    return hist


def _make_scatter(NPAD, D, E, K, GB):
    EPW = E // NW
    nch = EPW // K
    nblk = nch // GB
    RPT = NPAD // NS
    assert nch % GB == 0 and nblk % 2 == 0 and GB % 2 == 0

    @functools.partial(
        pl.kernel,
        out_type=jax.ShapeDtypeStruct((NC * NPAD, D), jnp.bfloat16),
        mesh=_sc_mesh(),
        compiler_params=pltpu.CompilerParams(use_tc_tiling_on_sc=False),
        scratch_types=[
            pltpu.VMEM_SHARED((NPAD, D), jnp.bfloat16),
            pltpu.VMEM((GB, K), jnp.int32),
            pltpu.VMEM((GB, K), jnp.int32),
            pltpu.VMEM((nch, K), jnp.int32),
            [pltpu.VMEM((K, D), jnp.bfloat16)] * 8,
            [pltpu.SemaphoreType.DMA] * 8,
            [pltpu.SemaphoreType.DMA] * 8,
            pltpu.SemaphoreType.DMA,
            pltpu.SemaphoreType.DMA,
        ],
    )
    def scat(gbf_hbm, src_hbm, dst_hbm, out_hbm,
             acc_sh, ixa, ixb, dsta, rows, gsem, ssem, sem_ia, sem_ib):
        c = lax.axis_index("c")
        s = lax.axis_index("s")
        w = c * NS + s
        pltpu.sync_copy(dst_hbm.at[w], dsta)
        pltpu.sync_copy(gbf_hbm.at[pl.ds(s * RPT, RPT)],
                        acc_sh.at[pl.ds(s * RPT, RPT)])
        plsc.subcore_barrier()

        def stage(b, buf, sem):
            return pltpu.async_copy(
                src_hbm.at[w, pl.ds(b * GB, GB)], buf, sem)

        def wait_stage(buf, sem):
            pltpu.make_async_copy(src_hbm.at[w, pl.ds(0, GB)], buf, sem).wait()

        def gather(ix, t, buf, sem):
            pltpu.async_copy(gbf_hbm.at[ix.at[t]], buf, sem)

        def wait_rows(buf, sem):
            pltpu.make_async_copy(gbf_hbm.at[ixa.at[0]], buf, sem).wait()

        def scatter(j, buf, sem):
            pltpu.async_copy(buf, acc_sh.at[dsta.at[j]], sem, add=True)

        def wait_scat(buf, sem):
            pltpu.make_async_copy(buf, acc_sh.at[dsta.at[0]], sem).wait()

        DEPTH = 4
        NB = 2 * DEPTH

        def step(jglob, ix, t, gx, gt, has_g, q):
            q2 = (q + DEPTH) % NB
            wait_rows(rows[q], gsem[q])
            scatter(jglob, rows[q], ssem[q])

            @pl.when(jglob >= DEPTH)
            def _():
                wait_scat(rows[q2], ssem[q2])

            @pl.when(has_g)
            def _():
                gather(gx, gt, rows[q2], gsem[q2])

        def block(b, ix, nxt_ix, nxt_sem, has_next):
            for t in range(DEPTH):
                step(b * GB + t, ix, t, ix, t + DEPTH, jnp.bool_(True), t)

            @pl.when(has_next)
            def _():
                wait_stage(nxt_ix, nxt_sem)

            for t in range(DEPTH, NB):
                step(b * GB + t, ix, t, nxt_ix, t - DEPTH, has_next, t)

        stage(0, ixa, sem_ia).wait()
        stage(1, ixb, sem_ib)
        for t in range(DEPTH):
            gather(ixa, t, rows[t], gsem[t])

        def outer2(bi, carry):
            b0 = 2 * bi
            block(b0, ixa, ixb, sem_ib, b0 + 1 < nblk)

            @pl.when(b0 + 2 < nblk)
            def _():
                stage(b0 + 2, ixa, sem_ia)

            block(b0 + 1, ixb, ixa, sem_ia, b0 + 2 < nblk)

            @pl.when(b0 + 3 < nblk)
            def _():
                stage(b0 + 3, ixb, sem_ib)

            return carry

        lax.fori_loop(0, nblk // 2, outer2, 0)
        for j in range(nch - DEPTH, nch):
            wait_scat(rows[j % NB], ssem[j % NB])
        plsc.subcore_barrier()
        pltpu.sync_copy(acc_sh.at[pl.ds(s * RPT, RPT)],
                        out_hbm.at[pl.ds(c * NPAD + s * RPT, RPT)])

    return scat


def _matmul_scale(x, W, d0, d1, NPAD):
    N, Din = x.shape
    Dout = W.shape[1]
    BN = 1000

    def body(x_ref, w_ref, d0_ref, d1_ref, gbf_ref, dinv_ref):
        dinv = lax.rsqrt(d0_ref[...] + d1_ref[...] + 1.0)
        h = jnp.dot(x_ref[...], w_ref[...],
                    preferred_element_type=jnp.float32)
        gbf_ref[...] = (h * dinv).astype(jnp.bfloat16)
        dinv_ref[...] = dinv

    return pl.pallas_call(
        body,
        grid=(N // BN,),
        in_specs=[
            pl.BlockSpec((BN, Din), lambda i: (i, 0)),
            pl.BlockSpec((Din, Dout), lambda i: (0, 0)),
            pl.BlockSpec((BN, 1), lambda i: (i, 0)),
            pl.BlockSpec((BN, 1), lambda i: (i, 0)),
        ],
        out_specs=[
            pl.BlockSpec((BN, Dout), lambda i: (i, 0)),
            pl.BlockSpec((BN, 1), lambda i: (i, 0)),
        ],
        out_shape=[
            jax.ShapeDtypeStruct((NPAD, Dout), jnp.bfloat16),
            jax.ShapeDtypeStruct((N, 1), jnp.float32),
        ],
    )(x, W, d0, d1)


def _final(acc, g, dinv, b2d, N):
    D = g.shape[1]
    BN = 1000

    def body(a_ref, g_ref, dinv_ref, b_ref, o_ref):
        a = a_ref[...].astype(jnp.float32)
        o_ref[...] = (dinv_ref[...]
                      * (a[0] + a[1] - g_ref[...].astype(jnp.float32))
                      + b_ref[...])

    return pl.pallas_call(
        body,
        grid=(N // BN,),
        in_specs=[
            pl.BlockSpec((2, BN, D), lambda i: (0, i, 0)),
            pl.BlockSpec((BN, D), lambda i: (i, 0)),
            pl.BlockSpec((BN, 1), lambda i: (i, 0)),
            pl.BlockSpec((1, D), lambda i: (0, 0)),
        ],
        out_specs=pl.BlockSpec((BN, D), lambda i: (i, 0)),
        out_shape=jax.ShapeDtypeStruct((N, D), jnp.float32),
    )(acc, g, dinv, b2d)


def kernel(x, edge_index, t_embed, W, b):
    N, Din = x.shape
    Dout = W.shape[1]
    E = edge_index.shape[1]
    src = edge_index[0]
    dst = edge_index[1]

    NPAD = 10240
    KH = 125
    K = 125
    GB = 8

    nch = E // NW // K
    src3d = src.reshape(NW, nch, K)
    dst3d = dst.reshape(NW, nch, K)

    degp = _make_hist(E, NPAD, KH)(dst3d)
    d0 = degp[:N].reshape(N, 1)
    d1 = degp[NPAD:NPAD + N].reshape(N, 1)

    gbf, dinv = _matmul_scale(x, W, d0, d1, NPAD)

    acc = _make_scatter(NPAD, Dout, E, K, GB)(gbf, src3d, dst3d)
    acc = acc.reshape(NC, NPAD, Dout)

    out = _final(acc, gbf, dinv, b.reshape(1, Dout), N)
    return (out, edge_index, t_embed)


# speedup vs baseline: 1.3608x; 1.0293x over previous
import functools

import jax
import jax.numpy as jnp
from jax import lax
from jax.experimental import pallas as pl
from jax.experimental.pallas import tpu as pltpu
from jax.experimental.pallas import tpu_sc as plsc

NC = 2
NS = 16
NW = NC * NS


def _sc_mesh():
    return plsc.VectorSubcoreMesh(
        core_axis_name="c", subcore_axis_name="s",
        num_cores=NC, num_subcores=NS)


def _make_hist(E, MDEG, K):
    EPW = E // NW
    nch = EPW // K
    RPT = MDEG // NS

    KUP = (K + 15) // 16 * 16

    @functools.partial(
        pl.kernel,
        out_type=jax.ShapeDtypeStruct((NC * MDEG,), jnp.float32),
        mesh=_sc_mesh(),
        compiler_params=pltpu.CompilerParams(use_tc_tiling_on_sc=False),
        scratch_types=[
            pltpu.VMEM_SHARED((MDEG,), jnp.float32),
            pltpu.VMEM((nch, K), jnp.int32),
            pltpu.VMEM((KUP,), jnp.float32),
            pltpu.VMEM((RPT,), jnp.float32),
            [pltpu.SemaphoreType.DMA] * 4,
        ],
    )
    def hist(dst_hbm, out_hbm, deg_sh, dste, ones_v, zbuf, ssem):
        c = lax.axis_index("c")
        s = lax.axis_index("s")
        w = c * NS + s
        for i in range(RPT // 16):
            zbuf[pl.ds(i * 16, 16)] = jnp.zeros((16,), jnp.float32)
        for i in range(KUP // 16):
            ones_v[pl.ds(i * 16, 16)] = jnp.ones((16,), jnp.float32)
        pltpu.sync_copy(zbuf, deg_sh.at[pl.ds(s * RPT, RPT)])
        pltpu.sync_copy(dst_hbm.at[w], dste)
        plsc.subcore_barrier()

        def scat1(j, sem):
            pltpu.async_copy(ones_v.at[pl.ds(0, K)],
                             deg_sh.at[dste.at[j]], sem, add=True)

        def drain(sem):
            pltpu.make_async_copy(ones_v.at[pl.ds(0, K)],
                                  deg_sh.at[dste.at[0]], sem).wait()

        def body(ti, carry):
            for k in range(4):
                j = 4 * ti + k

                @pl.when(j >= 4)
                def _():
                    drain(ssem[k])

                scat1(j, ssem[k])
            return carry

        lax.fori_loop(0, nch // 4, body, 0)
        for k in range(4):
            drain(ssem[k])
        plsc.subcore_barrier()
        pltpu.sync_copy(deg_sh.at[pl.ds(s * RPT, RPT)],
                        out_hbm.at[pl.ds(c * MDEG + s * RPT, RPT)])

    return hist


def _make_scatter(NPAD, D, E, K, GB):
    EPW = E // NW
    nch = EPW // K
    nblk = nch // GB
    RPT = NPAD // NS
    assert nch % GB == 0 and nblk % 2 == 0 and GB % 2 == 0

    @functools.partial(
        pl.kernel,
        out_type=jax.ShapeDtypeStruct((NC * NPAD, D), jnp.bfloat16),
        mesh=_sc_mesh(),
        compiler_params=pltpu.CompilerParams(use_tc_tiling_on_sc=False),
        scratch_types=[
            pltpu.VMEM_SHARED((NPAD, D), jnp.bfloat16),
            pltpu.VMEM((GB, K), jnp.int32),
            pltpu.VMEM((GB, K), jnp.int32),
            pltpu.VMEM((nch, K), jnp.int32),
            [pltpu.VMEM((K, D), jnp.bfloat16)] * 8,
            [pltpu.SemaphoreType.DMA] * 8,
            [pltpu.SemaphoreType.DMA] * 8,
            pltpu.SemaphoreType.DMA,
            pltpu.SemaphoreType.DMA,
        ],
    )
    def scat(gbf_hbm, src_hbm, dst_hbm, out_hbm,
             acc_sh, ixa, ixb, dsta, rows, gsem, ssem, sem_ia, sem_ib):
        c = lax.axis_index("c")
        s = lax.axis_index("s")
        w = c * NS + s
        pltpu.sync_copy(dst_hbm.at[w], dsta)
        pltpu.sync_copy(gbf_hbm.at[pl.ds(s * RPT, RPT)],
                        acc_sh.at[pl.ds(s * RPT, RPT)])
        plsc.subcore_barrier()

        def stage(b, buf, sem):
            return pltpu.async_copy(
                src_hbm.at[w, pl.ds(b * GB, GB)], buf, sem)

        def wait_stage(buf, sem):
            pltpu.make_async_copy(src_hbm.at[w, pl.ds(0, GB)], buf, sem).wait()

        def gather(ix, t, buf, sem):
            pltpu.async_copy(gbf_hbm.at[ix.at[t]], buf, sem)

        def wait_rows(buf, sem):
            pltpu.make_async_copy(gbf_hbm.at[ixa.at[0]], buf, sem).wait()

        def scatter(j, buf, sem):
            pltpu.async_copy(buf, acc_sh.at[dsta.at[j]], sem, add=True)

        def wait_scat(buf, sem):
            pltpu.make_async_copy(buf, acc_sh.at[dsta.at[0]], sem).wait()

        DEPTH = 4
        NB = 2 * DEPTH

        def step(jglob, ix, t, gx, gt, has_g, q):
            q2 = (q + DEPTH) % NB
            wait_rows(rows[q], gsem[q])
            scatter(jglob, rows[q], ssem[q])

            @pl.when(jglob >= DEPTH)
            def _():
                wait_scat(rows[q2], ssem[q2])

            @pl.when(has_g)
            def _():
                gather(gx, gt, rows[q2], gsem[q2])

        def block(b, ix, nxt_ix, nxt_sem, has_next):
            for t in range(DEPTH):
                step(b * GB + t, ix, t, ix, t + DEPTH, jnp.bool_(True), t)

            @pl.when(has_next)
            def _():
                wait_stage(nxt_ix, nxt_sem)

            for t in range(DEPTH, NB):
                step(b * GB + t, ix, t, nxt_ix, t - DEPTH, has_next, t)

        stage(0, ixa, sem_ia).wait()
        stage(1, ixb, sem_ib)
        for t in range(DEPTH):
            gather(ixa, t, rows[t], gsem[t])

        def outer2(bi, carry):
            b0 = 2 * bi
            block(b0, ixa, ixb, sem_ib, b0 + 1 < nblk)

            @pl.when(b0 + 2 < nblk)
            def _():
                stage(b0 + 2, ixa, sem_ia)

            block(b0 + 1, ixb, ixa, sem_ia, b0 + 2 < nblk)

            @pl.when(b0 + 3 < nblk)
            def _():
                stage(b0 + 3, ixb, sem_ib)

            return carry

        lax.fori_loop(0, nblk // 2, outer2, 0)
        for j in range(nch - DEPTH, nch):
            wait_scat(rows[j % NB], ssem[j % NB])
        plsc.subcore_barrier()
        pltpu.sync_copy(acc_sh.at[pl.ds(s * RPT, RPT)],
                        out_hbm.at[pl.ds(c * NPAD + s * RPT, RPT)])

    return scat


def _matmul_scale(x, W, d0, d1, NPAD):
    N, Din = x.shape
    Dout = W.shape[1]
    BN = 1000

    def body(x_ref, w_ref, d0_ref, d1_ref, gbf_ref, dinv_ref):
        dinv = lax.rsqrt(d0_ref[...] + d1_ref[...] + 1.0)
        h = jnp.dot(x_ref[...], w_ref[...],
                    preferred_element_type=jnp.float32)
        gbf_ref[...] = (h * dinv).astype(jnp.bfloat16)
        dinv_ref[...] = dinv

    return pl.pallas_call(
        body,
        grid=(N // BN,),
        in_specs=[
            pl.BlockSpec((BN, Din), lambda i: (i, 0)),
            pl.BlockSpec((Din, Dout), lambda i: (0, 0)),
            pl.BlockSpec((BN, 1), lambda i: (i, 0)),
            pl.BlockSpec((BN, 1), lambda i: (i, 0)),
        ],
        out_specs=[
            pl.BlockSpec((BN, Dout), lambda i: (i, 0)),
            pl.BlockSpec((BN, 1), lambda i: (i, 0)),
        ],
        out_shape=[
            jax.ShapeDtypeStruct((NPAD, Dout), jnp.bfloat16),
            jax.ShapeDtypeStruct((N, 1), jnp.float32),
        ],
    )(x, W, d0, d1)


def _final(acc, g, dinv, b2d, N):
    D = g.shape[1]
    BN = 1000

    def body(a_ref, g_ref, dinv_ref, b_ref, o_ref):
        a = a_ref[...].astype(jnp.float32)
        o_ref[...] = (dinv_ref[...]
                      * (a[0] + a[1] - g_ref[...].astype(jnp.float32))
                      + b_ref[...])

    return pl.pallas_call(
        body,
        grid=(N // BN,),
        in_specs=[
            pl.BlockSpec((2, BN, D), lambda i: (0, i, 0)),
            pl.BlockSpec((BN, D), lambda i: (i, 0)),
            pl.BlockSpec((BN, 1), lambda i: (i, 0)),
            pl.BlockSpec((1, D), lambda i: (0, 0)),
        ],
        out_specs=pl.BlockSpec((BN, D), lambda i: (i, 0)),
        out_shape=jax.ShapeDtypeStruct((N, D), jnp.float32),
    )(acc, g, dinv, b2d)


def kernel(x, edge_index, t_embed, W, b):
    N, Din = x.shape
    Dout = W.shape[1]
    E = edge_index.shape[1]
    src = edge_index[0]
    dst = edge_index[1]

    NPAD = 10240
    KH = 125
    K = 125
    GB = 8

    nch = E // NW // K
    src3d = src.reshape(NW, nch, K)
    dst3d = dst.reshape(NW, nch, K)

    degp = _make_hist(E, NPAD, KH)(dst3d)
    d0 = degp[:N].reshape(N, 1)
    d1 = degp[NPAD:NPAD + N].reshape(N, 1)

    gbf, dinv = _matmul_scale(x, W, d0, d1, NPAD)

    acc = _make_scatter(NPAD, Dout, E, K, GB)(gbf, src3d, dst3d)
    acc = acc.reshape(NC, NPAD, Dout)

    out = _final(acc, gbf, dinv, b.reshape(1, Dout), N)
    return (out, edge_index, t_embed)
